# Initial kernel scaffold; baseline (speedup 1.0000x reference)
import jax
import jax.numpy as jnp
from jax import lax
from jax.experimental import pallas as pl
from jax.experimental.pallas import tpu as pltpu
from jax.experimental.pallas import tpu_sc as plsc

NC, NS = 2, 16
N, E, D, DE = 10000, 320000, 128, 16
DEGW = 16
EPC = E // NC
EPT = EPC // NS
K = 400
NCHUNK = EPT // K
ZR = 624
ZREM = N - NS * ZR

_mesh = plsc.VectorSubcoreMesh(core_axis_name="c", subcore_axis_name="s",
                               num_cores=NC, num_subcores=NS)


def _row_ranges(s):
    r0 = s * ZR
    return [(r0, K), (r0 + K, ZR - K)]


def _make_segsum(with_deg):
    out_type = [jax.ShapeDtypeStruct((NC * N, D), jnp.float32)]
    scratch = [
        pltpu.VMEM((K,), jnp.int32),
        pltpu.VMEM((K,), jnp.int32),
        pltpu.VMEM((K, D), jnp.float32),
        pltpu.VMEM_SHARED((N, D), jnp.float32),
        pltpu.SemaphoreType.DMA,
    ]
    if with_deg:
        out_type.append(jax.ShapeDtypeStruct((NC * N, DEGW), jnp.float32))
        scratch += [
            pltpu.VMEM((K, DEGW), jnp.float32),
            pltpu.VMEM_SHARED((N, DEGW), jnp.float32),
        ]

    def body(*refs):
        if with_deg:
            (table, src, dst, zkd, zk16, ones16, agg_out, deg_out,
             sidx, didx, gbuf, acc, sem, obuf, dacc) = refs
        else:
            (table, src, dst, zkd, agg_out,
             sidx, didx, gbuf, acc, sem) = refs
        c = lax.axis_index("c")
        s = lax.axis_index("s")

        pltpu.sync_copy(zkd, gbuf)
        for off, sz in _row_ranges(s):
            pltpu.sync_copy(gbuf.at[pl.ds(0, sz)], acc.at[pl.ds(off, sz)])

        @pl.when(s == 0)
        def _():
            pltpu.sync_copy(gbuf.at[pl.ds(0, ZREM)],
                            acc.at[pl.ds(NS * ZR, ZREM)])

        if with_deg:
            pltpu.sync_copy(zk16, obuf)
            for off, sz in _row_ranges(s):
                pltpu.sync_copy(obuf.at[pl.ds(0, sz)], dacc.at[pl.ds(off, sz)])

            @pl.when(s == 0)
            def _():
                pltpu.sync_copy(obuf.at[pl.ds(0, ZREM)],
                                dacc.at[pl.ds(NS * ZR, ZREM)])

            pltpu.sync_copy(ones16, obuf)

        plsc.subcore_barrier()

        base_e = (c * NS + s) * EPT

        def chunk(j, carry):
            off = pl.multiple_of(base_e + j * K, 8)
            pltpu.sync_copy(src.at[pl.ds(off, K)], sidx)
            pltpu.sync_copy(dst.at[pl.ds(off, K)], didx)
            pltpu.async_copy(table.at[sidx], gbuf, sem).wait()
            pltpu.sync_copy(gbuf, acc.at[didx], add=True)
            if with_deg:
                pltpu.sync_copy(obuf, dacc.at[didx], add=True)
            return carry

        lax.fori_loop(0, NCHUNK, chunk, 0)

        plsc.subcore_barrier()

        out_base = c * N
        for off, sz in _row_ranges(s):
            pltpu.sync_copy(acc.at[pl.ds(off, sz)],
                            agg_out.at[pl.ds(out_base + off, sz)])
            if with_deg:
                pltpu.sync_copy(dacc.at[pl.ds(off, sz)],
                                deg_out.at[pl.ds(out_base + off, sz)])

        @pl.when(s == 0)
        def _():
            pltpu.sync_copy(acc.at[pl.ds(NS * ZR, ZREM)],
                            agg_out.at[pl.ds(out_base + NS * ZR, ZREM)])
            if with_deg:
                pltpu.sync_copy(dacc.at[pl.ds(NS * ZR, ZREM)],
                                deg_out.at[pl.ds(out_base + NS * ZR, ZREM)])

    return pl.kernel(body, out_type=tuple(out_type), mesh=_mesh,
                     scratch_types=tuple(scratch))


_segsum_deg = _make_segsum(True)
_segsum = _make_segsum(False)


def _gpair_body(A, B, u, v, gu_out, gv_out, uidx, vidx, gbufa, gbufb,
                sema, semb):
    c = lax.axis_index("c")
    s = lax.axis_index("s")
    base_e = (c * NS + s) * EPT

    def chunk(j, carry):
        off = pl.multiple_of(base_e + j * K, 8)
        pltpu.sync_copy(u.at[pl.ds(off, K)], uidx)
        pltpu.sync_copy(v.at[pl.ds(off, K)], vidx)
        cpa = pltpu.async_copy(A.at[uidx], gbufa, sema)
        cpb = pltpu.async_copy(B.at[vidx], gbufb, semb)
        cpa.wait()
        cpb.wait()
        pltpu.sync_copy(gbufa, gu_out.at[pl.ds(off, K)])
        pltpu.sync_copy(gbufb, gv_out.at[pl.ds(off, K)])
        return carry

    lax.fori_loop(0, NCHUNK, chunk, 0)


_gpair = pl.kernel(
    _gpair_body,
    out_type=(jax.ShapeDtypeStruct((E, D), jnp.float32),
              jax.ShapeDtypeStruct((E, D), jnp.float32)),
    mesh=_mesh,
    scratch_types=(
        pltpu.VMEM((K,), jnp.int32),
        pltpu.VMEM((K,), jnp.int32),
        pltpu.VMEM((K, D), jnp.float32),
        pltpu.VMEM((K, D), jnp.float32),
        pltpu.SemaphoreType.DMA,
        pltpu.SemaphoreType.DMA,
    ),
)


def _dt(p, w):
    return lax.dot_general(p, w, (((1,), (1,)), ((), ())),
                           preferred_element_type=jnp.float32)


BR = 1000


def _tc1_body(a0, a1, d0, d1, xr, wl, bl, wr, out):
    d = d0[:, 0:1] + d1[:, 0:1]
    mean = (a0[...] + a1[...]) / jnp.maximum(d, 1.0)
    out[...] = jnp.maximum(_dt(mean, wl[...]) + _dt(xr[...], wr[...])
                           + bl[...], 0.0)


def _tc2_body(a0, a1, d0, d1, hr, wl, bl, wr, wu, wv, aout, bout):
    d = d0[:, 0:1] + d1[:, 0:1]
    mean = (a0[...] + a1[...]) / jnp.maximum(d, 1.0)
    h = jnp.maximum(_dt(mean, wl[...]) + _dt(hr[...], wr[...]) + bl[...], 0.0)
    aout[...] = _dt(h, wu[...])
    bout[...] = _dt(h, wv[...])


def _agg_specs():
    return [
        pl.BlockSpec((BR, D), lambda i: (i, 0)),
        pl.BlockSpec((BR, D), lambda i: (i + N // BR, 0)),
        pl.BlockSpec((BR, DEGW), lambda i: (i, 0)),
        pl.BlockSpec((BR, DEGW), lambda i: (i + N // BR, 0)),
        pl.BlockSpec((BR, D), lambda i: (i, 0)),
        pl.BlockSpec((D, D), lambda i: (0, 0)),
        pl.BlockSpec((1, D), lambda i: (0, 0)),
        pl.BlockSpec((D, D), lambda i: (0, 0)),
    ]


def _tc_layer1(aggp, degp, x, wl, bl, wr):
    return pl.pallas_call(
        _tc1_body,
        grid=(N // BR,),
        in_specs=_agg_specs(),
        out_specs=pl.BlockSpec((BR, D), lambda i: (i, 0)),
        out_shape=jax.ShapeDtypeStruct((N, D), jnp.float32),
    )(aggp, aggp, degp, degp, x, wl, bl, wr)


def _tc_layer2(aggp, degp, h1, wl, bl, wr, wu, wv):
    return pl.pallas_call(
        _tc2_body,
        grid=(N // BR,),
        in_specs=_agg_specs() + [
            pl.BlockSpec((D, D), lambda i: (0, 0)),
            pl.BlockSpec((D, D), lambda i: (0, 0)),
        ],
        out_specs=[pl.BlockSpec((BR, D), lambda i: (i, 0)),
                   pl.BlockSpec((BR, D), lambda i: (i, 0))],
        out_shape=[jax.ShapeDtypeStruct((N, D), jnp.float32),
                   jax.ShapeDtypeStruct((N, D), jnp.float32)],
    )(aggp, aggp, degp, degp, h1, wl, bl, wr, wu, wv)


BE = 3200


def _tc3_body(gu, gv, at, le, w1e, b1, w2, b2, out):
    hmid = jnp.maximum(gu[...] + gv[...] + _dt(at[...], w1e[...]) + b1[...],
                       0.0)
    out[...] = _dt(hmid, w2[...]) + le[...] + b2[...]


def _tc_edge(gu, gv, attr, le, w1e, b1, w2, b2):
    return pl.pallas_call(
        _tc3_body,
        grid=(E // BE,),
        in_specs=[
            pl.BlockSpec((BE, D), lambda i: (i, 0)),
            pl.BlockSpec((BE, D), lambda i: (i, 0)),
            pl.BlockSpec((BE, DE), lambda i: (i, 0)),
            pl.BlockSpec((BE, 1), lambda i: (i, 0)),
            pl.BlockSpec((D, DE), lambda i: (0, 0)),
            pl.BlockSpec((1, D), lambda i: (0, 0)),
            pl.BlockSpec((1, D), lambda i: (0, 0)),
            pl.BlockSpec((1, 1), lambda i: (0, 0)),
        ],
        out_specs=pl.BlockSpec((BE, 1), lambda i: (i, 0)),
        out_shape=jax.ShapeDtypeStruct((E, 1), jnp.float32),
    )(gu, gv, attr, le, w1e, b1, w2, b2)


def kernel(x, edge_index, edge_u, edge_v, edge_attr, log_exposure,
           W1_l, b1_l, W1_r, W2_l, b2_l, W2_r, mW1, mb1, mW2, mb2):
    src = edge_index[0].astype(jnp.int32)
    dst = edge_index[1].astype(jnp.int32)
    u = edge_u.astype(jnp.int32)
    v = edge_v.astype(jnp.int32)
    zkd = jnp.zeros((K, D), jnp.float32)
    zk16 = jnp.zeros((K, DEGW), jnp.float32)
    ones16 = jnp.ones((K, DEGW), jnp.float32)

    aggp, degp = _segsum_deg(x, src, dst, zkd, zk16, ones16)
    h1 = _tc_layer1(aggp, degp, x, W1_l, b1_l.reshape(1, D), W1_r)
    (aggp2,) = _segsum(h1, src, dst, zkd)
    A, B = _tc_layer2(aggp2, degp, h1, W2_l, b2_l.reshape(1, D), W2_r,
                      mW1[:, :D], mW1[:, D:2 * D])
    gu, gv = _gpair(A, B, u, v)
    out = _tc_edge(gu, gv, edge_attr, log_exposure.reshape(E, 1),
                   mW1[:, 2 * D:], mb1.reshape(1, D), mW2.reshape(1, D),
                   mb2.reshape(1, 1))
    return out.reshape(E)


# speedup vs baseline: 3.4068x; 3.4068x over previous
import jax
import jax.numpy as jnp
from jax import lax
from jax.experimental import pallas as pl
from jax.experimental.pallas import tpu as pltpu
from jax.experimental.pallas import tpu_sc as plsc

NC, NS = 2, 16
N, E, D, DE = 10000, 320000, 128, 16
DEGW = 16
EPC = E // NC
EPT = EPC // NS
KS = 80
NCHUNKS = EPT // KS
KG = 400
NCHUNKG = EPT // KG
ZR = 624
ZREM = N - NS * ZR

_mesh = plsc.VectorSubcoreMesh(core_axis_name="c", subcore_axis_name="s",
                               num_cores=NC, num_subcores=NS)


def _row_ranges(s):
    r0 = s * ZR
    return [(r0 + i * KS, min(KS, ZR - i * KS))
            for i in range((ZR + KS - 1) // KS)]


EPT1 = E // NS
NCHUNK1 = EPT1 // KS


def _agg_deg_body(table, src, dst, zkd, onesd, agg_out, deg_out,
                  sidx, didx, gbuf, acc, sem):
    c = lax.axis_index("c")
    s = lax.axis_index("s")

    pltpu.sync_copy(zkd, gbuf)
    for off, sz in _row_ranges(s):
        pltpu.sync_copy(gbuf.at[pl.ds(0, sz)], acc.at[pl.ds(off, sz)])

    @pl.when(s == 0)
    def _():
        pltpu.sync_copy(gbuf.at[pl.ds(0, ZREM)], acc.at[pl.ds(NS * ZR, ZREM)])

    @pl.when(c == 1)
    def _():
        pltpu.sync_copy(onesd, gbuf)

    plsc.subcore_barrier()

    base_e = s * EPT1

    @pl.when(c == 0)
    def _():
        def chunk(j, carry):
            off = pl.multiple_of(base_e + j * KS, 8)
            pltpu.sync_copy(src.at[pl.ds(off, KS)], sidx)
            pltpu.sync_copy(dst.at[pl.ds(off, KS)], didx)
            pltpu.async_copy(table.at[sidx], gbuf, sem).wait()
            pltpu.sync_copy(gbuf, acc.at[didx], add=True)
            return carry
        lax.fori_loop(0, NCHUNK1, chunk, 0)

    @pl.when(c == 1)
    def _():
        def chunk(j, carry):
            off = pl.multiple_of(base_e + j * KS, 8)
            pltpu.sync_copy(dst.at[pl.ds(off, KS)], didx)
            pltpu.sync_copy(gbuf, acc.at[didx], add=True)
            return carry
        lax.fori_loop(0, NCHUNK1, chunk, 0)

    plsc.subcore_barrier()

    r0 = s * ZR

    @pl.when(c == 0)
    def _():
        pltpu.sync_copy(acc.at[pl.ds(r0, ZR)], agg_out.at[pl.ds(r0, ZR)])

    @pl.when(c == 1)
    def _():
        pltpu.sync_copy(acc.at[pl.ds(r0, ZR)], deg_out.at[pl.ds(r0, ZR)])

    @pl.when((s == 0) & (c == 0))
    def _():
        pltpu.sync_copy(acc.at[pl.ds(NS * ZR, ZREM)],
                        agg_out.at[pl.ds(NS * ZR, ZREM)])

    @pl.when((s == 0) & (c == 1))
    def _():
        pltpu.sync_copy(acc.at[pl.ds(NS * ZR, ZREM)],
                        deg_out.at[pl.ds(NS * ZR, ZREM)])


_agg_deg = pl.kernel(
    _agg_deg_body,
    out_type=(jax.ShapeDtypeStruct((N, D), jnp.float32),
              jax.ShapeDtypeStruct((N, D), jnp.float32)),
    mesh=_mesh,
    scratch_types=(
        pltpu.VMEM((KS,), jnp.int32),
        pltpu.VMEM((KS,), jnp.int32),
        pltpu.VMEM((KS, D), jnp.float32),
        pltpu.VMEM_SHARED((N, D), jnp.float32),
        pltpu.SemaphoreType.DMA,
    ),
)


def _segsum_body(table, src, dst, zkd, agg_out, sidx, didx, gbuf, acc, sem):
    c = lax.axis_index("c")
    s = lax.axis_index("s")

    pltpu.sync_copy(zkd, gbuf)
    for off, sz in _row_ranges(s):
        pltpu.sync_copy(gbuf.at[pl.ds(0, sz)], acc.at[pl.ds(off, sz)])

    @pl.when(s == 0)
    def _():
        pltpu.sync_copy(gbuf.at[pl.ds(0, ZREM)], acc.at[pl.ds(NS * ZR, ZREM)])

    plsc.subcore_barrier()

    base_e = (c * NS + s) * EPT

    def chunk(j, carry):
        off = pl.multiple_of(base_e + j * KS, 8)
        pltpu.sync_copy(src.at[pl.ds(off, KS)], sidx)
        pltpu.sync_copy(dst.at[pl.ds(off, KS)], didx)
        pltpu.async_copy(table.at[sidx], gbuf, sem).wait()
        pltpu.sync_copy(gbuf, acc.at[didx], add=True)
        return carry

    lax.fori_loop(0, NCHUNKS, chunk, 0)

    plsc.subcore_barrier()

    out_base = c * N
    r0 = s * ZR
    pltpu.sync_copy(acc.at[pl.ds(r0, ZR)], agg_out.at[pl.ds(out_base + r0, ZR)])

    @pl.when(s == 0)
    def _():
        pltpu.sync_copy(acc.at[pl.ds(NS * ZR, ZREM)],
                        agg_out.at[pl.ds(out_base + NS * ZR, ZREM)])


_segsum = pl.kernel(
    _segsum_body,
    out_type=(jax.ShapeDtypeStruct((NC * N, D), jnp.float32),),
    mesh=_mesh,
    scratch_types=(
        pltpu.VMEM((KS,), jnp.int32),
        pltpu.VMEM((KS,), jnp.int32),
        pltpu.VMEM((KS, D), jnp.float32),
        pltpu.VMEM_SHARED((N, D), jnp.float32),
        pltpu.SemaphoreType.DMA,
    ),
)


def _gpair_body(A, B, u, v, gu_out, gv_out, uidx, vidx, gbufa, gbufb,
                sema, semb):
    c = lax.axis_index("c")
    s = lax.axis_index("s")
    base_e = (c * NS + s) * EPT

    def chunk(j, carry):
        off = pl.multiple_of(base_e + j * KG, 8)
        pltpu.sync_copy(u.at[pl.ds(off, KG)], uidx)
        pltpu.sync_copy(v.at[pl.ds(off, KG)], vidx)
        cpa = pltpu.async_copy(A.at[uidx], gbufa, sema)
        cpb = pltpu.async_copy(B.at[vidx], gbufb, semb)
        cpa.wait()
        cpb.wait()
        pltpu.sync_copy(gbufa, gu_out.at[pl.ds(off, KG)])
        pltpu.sync_copy(gbufb, gv_out.at[pl.ds(off, KG)])
        return carry

    lax.fori_loop(0, NCHUNKG, chunk, 0)


_gpair = pl.kernel(
    _gpair_body,
    out_type=(jax.ShapeDtypeStruct((E, D), jnp.float32),
              jax.ShapeDtypeStruct((E, D), jnp.float32)),
    mesh=_mesh,
    scratch_types=(
        pltpu.VMEM((KG,), jnp.int32),
        pltpu.VMEM((KG,), jnp.int32),
        pltpu.VMEM((KG, D), jnp.float32),
        pltpu.VMEM((KG, D), jnp.float32),
        pltpu.SemaphoreType.DMA,
        pltpu.SemaphoreType.DMA,
    ),
)


def _dt(p, w):
    return lax.dot_general(p, w, (((1,), (1,)), ((), ())),
                           preferred_element_type=jnp.float32)


BR = 1000


def _tc1_body(a, dg, xr, wl, bl, wr, out):
    mean = a[...] / jnp.maximum(dg[:, 0:1], 1.0)
    out[...] = jnp.maximum(_dt(mean, wl[...]) + _dt(xr[...], wr[...])
                           + bl[...], 0.0)


def _tc2_body(a0, a1, dg, hr, wl, bl, wr, wu, wv, aout, bout):
    mean = (a0[...] + a1[...]) / jnp.maximum(dg[:, 0:1], 1.0)
    h = jnp.maximum(_dt(mean, wl[...]) + _dt(hr[...], wr[...]) + bl[...], 0.0)
    aout[...] = _dt(h, wu[...])
    bout[...] = _dt(h, wv[...])


def _tc_layer1(agg, deg, x, wl, bl, wr):
    return pl.pallas_call(
        _tc1_body,
        grid=(N // BR,),
        in_specs=[
            pl.BlockSpec((BR, D), lambda i: (i, 0)),
            pl.BlockSpec((BR, D), lambda i: (i, 0)),
            pl.BlockSpec((BR, D), lambda i: (i, 0)),
            pl.BlockSpec((D, D), lambda i: (0, 0)),
            pl.BlockSpec((1, D), lambda i: (0, 0)),
            pl.BlockSpec((D, D), lambda i: (0, 0)),
        ],
        out_specs=pl.BlockSpec((BR, D), lambda i: (i, 0)),
        out_shape=jax.ShapeDtypeStruct((N, D), jnp.float32),
    )(agg, deg, x, wl, bl, wr)


def _tc_layer2(aggp, deg, h1, wl, bl, wr, wu, wv):
    return pl.pallas_call(
        _tc2_body,
        grid=(N // BR,),
        in_specs=[
            pl.BlockSpec((BR, D), lambda i: (i, 0)),
            pl.BlockSpec((BR, D), lambda i: (i + N // BR, 0)),
            pl.BlockSpec((BR, D), lambda i: (i, 0)),
            pl.BlockSpec((BR, D), lambda i: (i, 0)),
            pl.BlockSpec((D, D), lambda i: (0, 0)),
            pl.BlockSpec((1, D), lambda i: (0, 0)),
            pl.BlockSpec((D, D), lambda i: (0, 0)),
            pl.BlockSpec((D, D), lambda i: (0, 0)),
            pl.BlockSpec((D, D), lambda i: (0, 0)),
        ],
        out_specs=[pl.BlockSpec((BR, D), lambda i: (i, 0)),
                   pl.BlockSpec((BR, D), lambda i: (i, 0))],
        out_shape=[jax.ShapeDtypeStruct((N, D), jnp.float32),
                   jax.ShapeDtypeStruct((N, D), jnp.float32)],
    )(aggp, aggp, deg, h1, wl, bl, wr, wu, wv)


BE = 3200


def _tc3_body(gu, gv, at, le, w1e, b1, w2, b2, out):
    hmid = jnp.maximum(gu[...] + gv[...] + _dt(at[...], w1e[...]) + b1[...],
                       0.0)
    s = lax.dot_general(hmid, w2[...], (((1,), (0,)), ((), ())),
                        preferred_element_type=jnp.float32)
    out[...] = s + le[...] + b2[0]


def _tc_edge(gu, gv, attr, le, w1e, b1, w2, b2):
    return pl.pallas_call(
        _tc3_body,
        grid=(E // BE,),
        in_specs=[
            pl.BlockSpec((BE, D), lambda i: (i, 0)),
            pl.BlockSpec((BE, D), lambda i: (i, 0)),
            pl.BlockSpec((BE, DE), lambda i: (i, 0)),
            pl.BlockSpec((BE, 1), lambda i: (i, 0)),
            pl.BlockSpec((D, DE), lambda i: (0, 0)),
            pl.BlockSpec((1, D), lambda i: (0, 0)),
            pl.BlockSpec((D, 1), lambda i: (0, 0)),
            pl.BlockSpec(memory_space=pltpu.SMEM),
        ],
        out_specs=pl.BlockSpec((BE, 1), lambda i: (i, 0)),
        out_shape=jax.ShapeDtypeStruct((E, 1), jnp.float32),
    )(gu, gv, attr, le, w1e, b1, w2, b2)


def kernel(x, edge_index, edge_u, edge_v, edge_attr, log_exposure,
           W1_l, b1_l, W1_r, W2_l, b2_l, W2_r, mW1, mb1, mW2, mb2):
    src = edge_index[0].astype(jnp.int32)
    dst = edge_index[1].astype(jnp.int32)
    u = edge_u.astype(jnp.int32)
    v = edge_v.astype(jnp.int32)
    zkd = jnp.zeros((KS, D), jnp.float32)
    onesd = jnp.ones((KS, D), jnp.float32)

    agg, deg = _agg_deg(x, src, dst, zkd, onesd)
    h1 = _tc_layer1(agg, deg, x, W1_l, b1_l.reshape(1, D), W1_r)
    (aggp2,) = _segsum(h1, src, dst, zkd)
    A, B = _tc_layer2(aggp2, deg, h1, W2_l, b2_l.reshape(1, D), W2_r,
                      mW1[:, :D], mW1[:, D:2 * D])
    gu, gv = _gpair(A, B, u, v)
    out = _tc_edge(gu, gv, edge_attr, log_exposure.reshape(E, 1),
                   mW1[:, 2 * D:], mb1.reshape(1, D), mW2.reshape(D, 1),
                   mb2)
    return out.reshape(E)


# speedup vs baseline: 4.4630x; 1.3100x over previous
import jax
import jax.numpy as jnp
from jax import lax
from jax.experimental import pallas as pl
from jax.experimental.pallas import tpu as pltpu
from jax.experimental.pallas import tpu_sc as plsc

NC, NS = 2, 16
N, E, D, DE = 10000, 320000, 128, 16
DEGW = 16
EPC = E // NC
EPT = EPC // NS
KS = 80
NCHUNKS = EPT // KS
KG = 200
NCHUNKG = EPT // KG
ZR = 624
ZREM = N - NS * ZR

_mesh = plsc.VectorSubcoreMesh(core_axis_name="c", subcore_axis_name="s",
                               num_cores=NC, num_subcores=NS)


def _row_ranges(s):
    r0 = s * ZR
    return [(r0 + i * KS, min(KS, ZR - i * KS))
            for i in range((ZR + KS - 1) // KS)]


EPT1 = E // NS
NCHUNK1 = EPT1 // KS


def _start_gather(table, src, dst, base, j, sb, db, gb, sem):
    off = pl.multiple_of(base + j * KS, 8)
    pltpu.sync_copy(src.at[pl.ds(off, KS)], sb)
    pltpu.sync_copy(dst.at[pl.ds(off, KS)], db)
    pltpu.async_copy(table.at[sb], gb, sem)


def _finish_scatter(table, acc, db, gb, sem):
    pltpu.make_async_copy(table.at[pl.ds(0, KS)], gb, sem).wait()
    pltpu.sync_copy(gb, acc.at[db], add=True)


def _pipelined_segsum_loop(table, src, dst, acc, base, nchunk,
                           s0, d0, g0, sem0, s1, d1, g1, sem1):
    _start_gather(table, src, dst, base, 0, s0, d0, g0, sem0)

    def pair(jj, carry):
        b = 2 * jj + 1
        c = 2 * jj + 2
        _start_gather(table, src, dst, base, b, s1, d1, g1, sem1)
        _finish_scatter(table, acc, d0, g0, sem0)

        @pl.when(c < nchunk)
        def _():
            _start_gather(table, src, dst, base, c, s0, d0, g0, sem0)

        _finish_scatter(table, acc, d1, g1, sem1)
        return carry

    lax.fori_loop(0, nchunk // 2, pair, 0)
    if nchunk % 2 == 1:
        _finish_scatter(table, acc, d0, g0, sem0)


def _zero_acc(zkd, gbuf, acc, s):
    pltpu.sync_copy(zkd, gbuf)
    for off, sz in _row_ranges(s):
        pltpu.sync_copy(gbuf.at[pl.ds(0, sz)], acc.at[pl.ds(off, sz)])

    @pl.when(s == 0)
    def _():
        pltpu.sync_copy(gbuf.at[pl.ds(0, ZREM)], acc.at[pl.ds(NS * ZR, ZREM)])


def _agg_deg_body(table, src, dst, zkd, onesd, agg_out, deg_out,
                  s0, d0, g0, sem0, s1, d1, g1, sem1, acc):
    c = lax.axis_index("c")
    s = lax.axis_index("s")

    _zero_acc(zkd, g0, acc, s)

    @pl.when(c == 1)
    def _():
        pltpu.sync_copy(onesd, g0)

    plsc.subcore_barrier()

    base_e = s * EPT1

    @pl.when(c == 0)
    def _():
        _pipelined_segsum_loop(table, src, dst, acc, base_e, NCHUNK1,
                               s0, d0, g0, sem0, s1, d1, g1, sem1)

    @pl.when(c == 1)
    def _():
        def chunk(j, carry):
            off = pl.multiple_of(base_e + j * KS, 8)
            pltpu.sync_copy(dst.at[pl.ds(off, KS)], d0)
            pltpu.sync_copy(g0, acc.at[d0], add=True)
            return carry
        lax.fori_loop(0, NCHUNK1, chunk, 0)

    plsc.subcore_barrier()

    r0 = s * ZR

    @pl.when(c == 0)
    def _():
        pltpu.sync_copy(acc.at[pl.ds(r0, ZR)], agg_out.at[pl.ds(r0, ZR)])

    @pl.when(c == 1)
    def _():
        pltpu.sync_copy(acc.at[pl.ds(r0, ZR)], deg_out.at[pl.ds(r0, ZR)])

    @pl.when((s == 0) & (c == 0))
    def _():
        pltpu.sync_copy(acc.at[pl.ds(NS * ZR, ZREM)],
                        agg_out.at[pl.ds(NS * ZR, ZREM)])

    @pl.when((s == 0) & (c == 1))
    def _():
        pltpu.sync_copy(acc.at[pl.ds(NS * ZR, ZREM)],
                        deg_out.at[pl.ds(NS * ZR, ZREM)])


_SEG_SCRATCH = (
    pltpu.VMEM((KS,), jnp.int32),
    pltpu.VMEM((KS,), jnp.int32),
    pltpu.VMEM((KS, D), jnp.float32),
    pltpu.SemaphoreType.DMA,
    pltpu.VMEM((KS,), jnp.int32),
    pltpu.VMEM((KS,), jnp.int32),
    pltpu.VMEM((KS, D), jnp.float32),
    pltpu.SemaphoreType.DMA,
    pltpu.VMEM_SHARED((N, D), jnp.float32),
)

_agg_deg = pl.kernel(
    _agg_deg_body,
    out_type=(jax.ShapeDtypeStruct((N, D), jnp.float32),
              jax.ShapeDtypeStruct((N, D), jnp.float32)),
    mesh=_mesh,
    scratch_types=_SEG_SCRATCH,
)


def _segsum_body(table, src, dst, zkd, agg_out,
                 s0, d0, g0, sem0, s1, d1, g1, sem1, acc):
    c = lax.axis_index("c")
    s = lax.axis_index("s")

    _zero_acc(zkd, g0, acc, s)
    plsc.subcore_barrier()

    base_e = (c * NS + s) * EPT
    _pipelined_segsum_loop(table, src, dst, acc, base_e, NCHUNKS,
                           s0, d0, g0, sem0, s1, d1, g1, sem1)

    plsc.subcore_barrier()

    out_base = c * N
    r0 = s * ZR
    pltpu.sync_copy(acc.at[pl.ds(r0, ZR)], agg_out.at[pl.ds(out_base + r0, ZR)])

    @pl.when(s == 0)
    def _():
        pltpu.sync_copy(acc.at[pl.ds(NS * ZR, ZREM)],
                        agg_out.at[pl.ds(out_base + NS * ZR, ZREM)])


_segsum = pl.kernel(
    _segsum_body,
    out_type=(jax.ShapeDtypeStruct((NC * N, D), jnp.float32),),
    mesh=_mesh,
    scratch_types=_SEG_SCRATCH,
)


def _gpair_body(A, B, u, v, gu_out, gv_out,
                ua0, va0, ga0, gb0, semA0, semB0,
                ua1, va1, ga1, gb1, semA1, semB1):
    c = lax.axis_index("c")
    s = lax.axis_index("s")
    base_e = (c * NS + s) * EPT

    def start(j, ua, va, ga, gb, semA, semB):
        off = pl.multiple_of(base_e + j * KG, 8)
        pltpu.sync_copy(u.at[pl.ds(off, KG)], ua)
        pltpu.sync_copy(v.at[pl.ds(off, KG)], va)
        pltpu.async_copy(A.at[ua], ga, semA)
        pltpu.async_copy(B.at[va], gb, semB)

    def finish(j, ga, gb, semA, semB):
        off = pl.multiple_of(base_e + j * KG, 8)
        pltpu.make_async_copy(A.at[pl.ds(0, KG)], ga, semA).wait()
        pltpu.make_async_copy(B.at[pl.ds(0, KG)], gb, semB).wait()
        pltpu.sync_copy(ga, gu_out.at[pl.ds(off, KG)])
        pltpu.sync_copy(gb, gv_out.at[pl.ds(off, KG)])

    start(0, ua0, va0, ga0, gb0, semA0, semB0)

    def pair(jj, carry):
        b = 2 * jj + 1
        cch = 2 * jj + 2
        start(b, ua1, va1, ga1, gb1, semA1, semB1)
        finish(2 * jj, ga0, gb0, semA0, semB0)

        @pl.when(cch < NCHUNKG)
        def _():
            start(cch, ua0, va0, ga0, gb0, semA0, semB0)

        finish(b, ga1, gb1, semA1, semB1)
        return carry

    lax.fori_loop(0, NCHUNKG // 2, pair, 0)


_gpair = pl.kernel(
    _gpair_body,
    out_type=(jax.ShapeDtypeStruct((E, D), jnp.float32),
              jax.ShapeDtypeStruct((E, D), jnp.float32)),
    mesh=_mesh,
    scratch_types=(
        pltpu.VMEM((KG,), jnp.int32),
        pltpu.VMEM((KG,), jnp.int32),
        pltpu.VMEM((KG, D), jnp.float32),
        pltpu.VMEM((KG, D), jnp.float32),
        pltpu.SemaphoreType.DMA,
        pltpu.SemaphoreType.DMA,
        pltpu.VMEM((KG,), jnp.int32),
        pltpu.VMEM((KG,), jnp.int32),
        pltpu.VMEM((KG, D), jnp.float32),
        pltpu.VMEM((KG, D), jnp.float32),
        pltpu.SemaphoreType.DMA,
        pltpu.SemaphoreType.DMA,
    ),
)


def _dt(p, w):
    return lax.dot_general(p, w, (((1,), (1,)), ((), ())),
                           preferred_element_type=jnp.float32)


BR = 1000


def _tc1_body(a, dg, xr, wl, bl, wr, out):
    mean = a[...] / jnp.maximum(dg[:, 0:1], 1.0)
    out[...] = jnp.maximum(_dt(mean, wl[...]) + _dt(xr[...], wr[...])
                           + bl[...], 0.0)


def _tc2_body(a0, a1, dg, hr, wl, bl, wr, wu, wv, aout, bout):
    mean = (a0[...] + a1[...]) / jnp.maximum(dg[:, 0:1], 1.0)
    h = jnp.maximum(_dt(mean, wl[...]) + _dt(hr[...], wr[...]) + bl[...], 0.0)
    aout[...] = _dt(h, wu[...])
    bout[...] = _dt(h, wv[...])


def _tc_layer1(agg, deg, x, wl, bl, wr):
    return pl.pallas_call(
        _tc1_body,
        grid=(N // BR,),
        in_specs=[
            pl.BlockSpec((BR, D), lambda i: (i, 0)),
            pl.BlockSpec((BR, D), lambda i: (i, 0)),
            pl.BlockSpec((BR, D), lambda i: (i, 0)),
            pl.BlockSpec((D, D), lambda i: (0, 0)),
            pl.BlockSpec((1, D), lambda i: (0, 0)),
            pl.BlockSpec((D, D), lambda i: (0, 0)),
        ],
        out_specs=pl.BlockSpec((BR, D), lambda i: (i, 0)),
        out_shape=jax.ShapeDtypeStruct((N, D), jnp.float32),
    )(agg, deg, x, wl, bl, wr)


def _tc_layer2(aggp, deg, h1, wl, bl, wr, wu, wv):
    return pl.pallas_call(
        _tc2_body,
        grid=(N // BR,),
        in_specs=[
            pl.BlockSpec((BR, D), lambda i: (i, 0)),
            pl.BlockSpec((BR, D), lambda i: (i + N // BR, 0)),
            pl.BlockSpec((BR, D), lambda i: (i, 0)),
            pl.BlockSpec((BR, D), lambda i: (i, 0)),
            pl.BlockSpec((D, D), lambda i: (0, 0)),
            pl.BlockSpec((1, D), lambda i: (0, 0)),
            pl.BlockSpec((D, D), lambda i: (0, 0)),
            pl.BlockSpec((D, D), lambda i: (0, 0)),
            pl.BlockSpec((D, D), lambda i: (0, 0)),
        ],
        out_specs=[pl.BlockSpec((BR, D), lambda i: (i, 0)),
                   pl.BlockSpec((BR, D), lambda i: (i, 0))],
        out_shape=[jax.ShapeDtypeStruct((N, D), jnp.float32),
                   jax.ShapeDtypeStruct((N, D), jnp.float32)],
    )(aggp, aggp, deg, h1, wl, bl, wr, wu, wv)


BE = 3200


def _tc3_body(gu, gv, at, le, w1e, b1, w2, b2, out):
    hmid = jnp.maximum(gu[...] + gv[...] + _dt(at[...], w1e[...]) + b1[...],
                       0.0)
    s = lax.dot_general(hmid, w2[...], (((1,), (0,)), ((), ())),
                        preferred_element_type=jnp.float32)
    out[...] = s + le[...] + b2[0]


def _tc_edge(gu, gv, attr, le, w1e, b1, w2, b2):
    return pl.pallas_call(
        _tc3_body,
        grid=(E // BE,),
        in_specs=[
            pl.BlockSpec((BE, D), lambda i: (i, 0)),
            pl.BlockSpec((BE, D), lambda i: (i, 0)),
            pl.BlockSpec((BE, DE), lambda i: (i, 0)),
            pl.BlockSpec((BE, 1), lambda i: (i, 0)),
            pl.BlockSpec((D, DE), lambda i: (0, 0)),
            pl.BlockSpec((1, D), lambda i: (0, 0)),
            pl.BlockSpec((D, 1), lambda i: (0, 0)),
            pl.BlockSpec(memory_space=pltpu.SMEM),
        ],
        out_specs=pl.BlockSpec((BE, 1), lambda i: (i, 0)),
        out_shape=jax.ShapeDtypeStruct((E, 1), jnp.float32),
    )(gu, gv, attr, le, w1e, b1, w2, b2)


def kernel(x, edge_index, edge_u, edge_v, edge_attr, log_exposure,
           W1_l, b1_l, W1_r, W2_l, b2_l, W2_r, mW1, mb1, mW2, mb2):
    src = edge_index[0].astype(jnp.int32)
    dst = edge_index[1].astype(jnp.int32)
    u = edge_u.astype(jnp.int32)
    v = edge_v.astype(jnp.int32)
    zkd = jnp.zeros((KS, D), jnp.float32)
    onesd = jnp.ones((KS, D), jnp.float32)

    agg, deg = _agg_deg(x, src, dst, zkd, onesd)
    h1 = _tc_layer1(agg, deg, x, W1_l, b1_l.reshape(1, D), W1_r)
    (aggp2,) = _segsum(h1, src, dst, zkd)
    A, B = _tc_layer2(aggp2, deg, h1, W2_l, b2_l.reshape(1, D), W2_r,
                      mW1[:, :D], mW1[:, D:2 * D])
    gu, gv = _gpair(A, B, u, v)
    out = _tc_edge(gu, gv, edge_attr, log_exposure.reshape(E, 1),
                   mW1[:, 2 * D:], mb1.reshape(1, D), mW2.reshape(D, 1),
                   mb2)
    return out.reshape(E)


# speedup vs baseline: 4.8445x; 1.0855x over previous
import jax
import jax.numpy as jnp
from jax import lax
from jax.experimental import pallas as pl
from jax.experimental.pallas import tpu as pltpu
from jax.experimental.pallas import tpu_sc as plsc

NC, NS = 2, 16
N, E, D, DE = 10000, 320000, 128, 16
DEGW = 16
EPC = E // NC
EPT = EPC // NS
KS = 80
NCHUNKS = EPT // KS
KG = 200
NCHUNKG = EPT // KG
ZR = 624
ZREM = N - NS * ZR

_mesh = plsc.VectorSubcoreMesh(core_axis_name="c", subcore_axis_name="s",
                               num_cores=NC, num_subcores=NS)


def _row_ranges(s):
    r0 = s * ZR
    return [(r0 + i * KS, min(KS, ZR - i * KS))
            for i in range((ZR + KS - 1) // KS)]


EPT1 = E // NS
NCHUNK1 = EPT1 // KS


def _start_gather(table, src, dst, base, j, sb, db, gb, sem):
    off = pl.multiple_of(base + j * KS, 8)
    pltpu.sync_copy(src.at[pl.ds(off, KS)], sb)
    pltpu.sync_copy(dst.at[pl.ds(off, KS)], db)
    pltpu.async_copy(table.at[sb], gb, sem)


def _finish_scatter(table, acc, db, gb, sem):
    pltpu.make_async_copy(table.at[pl.ds(0, KS)], gb, sem).wait()
    pltpu.sync_copy(gb, acc.at[db], add=True)


def _pipelined_segsum_loop(table, src, dst, acc, base, nchunk,
                           s0, d0, g0, sem0, s1, d1, g1, sem1):
    _start_gather(table, src, dst, base, 0, s0, d0, g0, sem0)

    def pair(jj, carry):
        b = 2 * jj + 1
        c = 2 * jj + 2
        _start_gather(table, src, dst, base, b, s1, d1, g1, sem1)
        _finish_scatter(table, acc, d0, g0, sem0)

        @pl.when(c < nchunk)
        def _():
            _start_gather(table, src, dst, base, c, s0, d0, g0, sem0)

        _finish_scatter(table, acc, d1, g1, sem1)
        return carry

    lax.fori_loop(0, nchunk // 2, pair, 0)
    if nchunk % 2 == 1:
        _finish_scatter(table, acc, d0, g0, sem0)


def _zero_acc(zkd, gbuf, acc, s):
    pltpu.sync_copy(zkd, gbuf)
    for off, sz in _row_ranges(s):
        pltpu.sync_copy(gbuf.at[pl.ds(0, sz)], acc.at[pl.ds(off, sz)])

    @pl.when(s == 0)
    def _():
        pltpu.sync_copy(gbuf.at[pl.ds(0, ZREM)], acc.at[pl.ds(NS * ZR, ZREM)])


def _agg_deg_body(table, src, dst, zkd, onesd, agg_out, deg_out,
                  s0, d0, g0, sem0, s1, d1, g1, sem1, acc):
    c = lax.axis_index("c")
    s = lax.axis_index("s")

    _zero_acc(zkd, g0, acc, s)

    @pl.when(c == 1)
    def _():
        pltpu.sync_copy(onesd, g0)

    plsc.subcore_barrier()

    base_e = s * EPT1

    @pl.when(c == 0)
    def _():
        _pipelined_segsum_loop(table, src, dst, acc, base_e, NCHUNK1,
                               s0, d0, g0, sem0, s1, d1, g1, sem1)

    @pl.when(c == 1)
    def _():
        def chunk(j, carry):
            off = pl.multiple_of(base_e + j * KS, 8)
            pltpu.sync_copy(dst.at[pl.ds(off, KS)], d0)
            pltpu.sync_copy(g0, acc.at[d0], add=True)
            return carry
        lax.fori_loop(0, NCHUNK1, chunk, 0)

    plsc.subcore_barrier()

    r0 = s * ZR

    @pl.when(c == 0)
    def _():
        pltpu.sync_copy(acc.at[pl.ds(r0, ZR)], agg_out.at[pl.ds(r0, ZR)])

    @pl.when(c == 1)
    def _():
        pltpu.sync_copy(acc.at[pl.ds(r0, ZR)], deg_out.at[pl.ds(r0, ZR)])

    @pl.when((s == 0) & (c == 0))
    def _():
        pltpu.sync_copy(acc.at[pl.ds(NS * ZR, ZREM)],
                        agg_out.at[pl.ds(NS * ZR, ZREM)])

    @pl.when((s == 0) & (c == 1))
    def _():
        pltpu.sync_copy(acc.at[pl.ds(NS * ZR, ZREM)],
                        deg_out.at[pl.ds(NS * ZR, ZREM)])


_SEG_SCRATCH = (
    pltpu.VMEM((KS,), jnp.int32),
    pltpu.VMEM((KS,), jnp.int32),
    pltpu.VMEM((KS, D), jnp.float32),
    pltpu.SemaphoreType.DMA,
    pltpu.VMEM((KS,), jnp.int32),
    pltpu.VMEM((KS,), jnp.int32),
    pltpu.VMEM((KS, D), jnp.float32),
    pltpu.SemaphoreType.DMA,
    pltpu.VMEM_SHARED((N, D), jnp.float32),
)

_agg_deg = pl.kernel(
    _agg_deg_body,
    out_type=(jax.ShapeDtypeStruct((N, D), jnp.float32),
              jax.ShapeDtypeStruct((N, D), jnp.float32)),
    mesh=_mesh,
    scratch_types=_SEG_SCRATCH,
)


def _segsum_body(table, src, dst, zkd, agg_out,
                 s0, d0, g0, sem0, s1, d1, g1, sem1, acc):
    c = lax.axis_index("c")
    s = lax.axis_index("s")

    _zero_acc(zkd, g0, acc, s)
    plsc.subcore_barrier()

    base_e = (c * NS + s) * EPT
    _pipelined_segsum_loop(table, src, dst, acc, base_e, NCHUNKS,
                           s0, d0, g0, sem0, s1, d1, g1, sem1)

    plsc.subcore_barrier()

    out_base = c * N
    r0 = s * ZR
    pltpu.sync_copy(acc.at[pl.ds(r0, ZR)], agg_out.at[pl.ds(out_base + r0, ZR)])

    @pl.when(s == 0)
    def _():
        pltpu.sync_copy(acc.at[pl.ds(NS * ZR, ZREM)],
                        agg_out.at[pl.ds(out_base + NS * ZR, ZREM)])


_segsum = pl.kernel(
    _segsum_body,
    out_type=(jax.ShapeDtypeStruct((NC * N, D), jnp.float32),),
    mesh=_mesh,
    scratch_types=_SEG_SCRATCH,
)


def _gpair_body(A, B, u, v, g_out,
                ua0, va0, ga0, semA0, semB0,
                ua1, va1, ga1, semA1, semB1):
    c = lax.axis_index("c")
    s = lax.axis_index("s")
    base_e = (c * NS + s) * EPT

    def start_a(j, ua, va, ga, semA):
        off = pl.multiple_of(base_e + j * KG, 8)
        pltpu.sync_copy(u.at[pl.ds(off, KG)], ua)
        pltpu.sync_copy(v.at[pl.ds(off, KG)], va)
        pltpu.async_copy(A.at[ua], ga, semA)

    def add_b(va, ga, semA, semB):
        pltpu.make_async_copy(A.at[pl.ds(0, KG)], ga, semA).wait()
        pltpu.async_copy(B.at[va], ga, semB, add=True)

    def finish(j, ga, semB):
        off = pl.multiple_of(base_e + j * KG, 8)
        pltpu.make_async_copy(B.at[pl.ds(0, KG)], ga, semB).wait()
        pltpu.sync_copy(ga, g_out.at[pl.ds(off, KG)])

    start_a(0, ua0, va0, ga0, semA0)

    def pair(jj, carry):
        b = 2 * jj + 1
        cch = 2 * jj + 2
        start_a(b, ua1, va1, ga1, semA1)
        add_b(va0, ga0, semA0, semB0)
        finish(2 * jj, ga0, semB0)

        @pl.when(cch < NCHUNKG)
        def _():
            start_a(cch, ua0, va0, ga0, semA0)

        add_b(va1, ga1, semA1, semB1)
        finish(b, ga1, semB1)
        return carry

    lax.fori_loop(0, NCHUNKG // 2, pair, 0)


_gpair = pl.kernel(
    _gpair_body,
    out_type=(jax.ShapeDtypeStruct((E, D), jnp.float32),),
    mesh=_mesh,
    scratch_types=(
        pltpu.VMEM((KG,), jnp.int32),
        pltpu.VMEM((KG,), jnp.int32),
        pltpu.VMEM((KG, D), jnp.float32),
        pltpu.SemaphoreType.DMA,
        pltpu.SemaphoreType.DMA,
        pltpu.VMEM((KG,), jnp.int32),
        pltpu.VMEM((KG,), jnp.int32),
        pltpu.VMEM((KG, D), jnp.float32),
        pltpu.SemaphoreType.DMA,
        pltpu.SemaphoreType.DMA,
    ),
)


def _dt(p, w):
    return lax.dot_general(p, w, (((1,), (1,)), ((), ())),
                           preferred_element_type=jnp.float32)


BR = 1000


def _tc1_body(a, dg, xr, wl, bl, wr, out):
    mean = a[...] / jnp.maximum(dg[:, 0:1], 1.0)
    out[...] = jnp.maximum(_dt(mean, wl[...]) + _dt(xr[...], wr[...])
                           + bl[...], 0.0)


def _tc2_body(a0, a1, dg, hr, wl, bl, wr, wu, wv, aout, bout):
    mean = (a0[...] + a1[...]) / jnp.maximum(dg[:, 0:1], 1.0)
    h = jnp.maximum(_dt(mean, wl[...]) + _dt(hr[...], wr[...]) + bl[...], 0.0)
    aout[...] = _dt(h, wu[...])
    bout[...] = _dt(h, wv[...])


def _tc_layer1(agg, deg, x, wl, bl, wr):
    return pl.pallas_call(
        _tc1_body,
        grid=(N // BR,),
        in_specs=[
            pl.BlockSpec((BR, D), lambda i: (i, 0)),
            pl.BlockSpec((BR, D), lambda i: (i, 0)),
            pl.BlockSpec((BR, D), lambda i: (i, 0)),
            pl.BlockSpec((D, D), lambda i: (0, 0)),
            pl.BlockSpec((1, D), lambda i: (0, 0)),
            pl.BlockSpec((D, D), lambda i: (0, 0)),
        ],
        out_specs=pl.BlockSpec((BR, D), lambda i: (i, 0)),
        out_shape=jax.ShapeDtypeStruct((N, D), jnp.float32),
    )(agg, deg, x, wl, bl, wr)


def _tc_layer2(aggp, deg, h1, wl, bl, wr, wu, wv):
    return pl.pallas_call(
        _tc2_body,
        grid=(N // BR,),
        in_specs=[
            pl.BlockSpec((BR, D), lambda i: (i, 0)),
            pl.BlockSpec((BR, D), lambda i: (i + N // BR, 0)),
            pl.BlockSpec((BR, D), lambda i: (i, 0)),
            pl.BlockSpec((BR, D), lambda i: (i, 0)),
            pl.BlockSpec((D, D), lambda i: (0, 0)),
            pl.BlockSpec((1, D), lambda i: (0, 0)),
            pl.BlockSpec((D, D), lambda i: (0, 0)),
            pl.BlockSpec((D, D), lambda i: (0, 0)),
            pl.BlockSpec((D, D), lambda i: (0, 0)),
        ],
        out_specs=[pl.BlockSpec((BR, D), lambda i: (i, 0)),
                   pl.BlockSpec((BR, D), lambda i: (i, 0))],
        out_shape=[jax.ShapeDtypeStruct((N, D), jnp.float32),
                   jax.ShapeDtypeStruct((N, D), jnp.float32)],
    )(aggp, aggp, deg, h1, wl, bl, wr, wu, wv)


BE = 3200


def _tc3_body(g, at, le, w1e, b1, w2, b2, out):
    hmid = jnp.maximum(g[...] + _dt(at[...], w1e[...]) + b1[...], 0.0)
    s = lax.dot_general(hmid, w2[...], (((1,), (0,)), ((), ())),
                        preferred_element_type=jnp.float32)
    out[...] = s + le[...] + b2[0]


def _tc_edge(g, attr, le, w1e, b1, w2, b2):
    return pl.pallas_call(
        _tc3_body,
        grid=(E // BE,),
        in_specs=[
            pl.BlockSpec((BE, D), lambda i: (i, 0)),
            pl.BlockSpec((BE, DE), lambda i: (i, 0)),
            pl.BlockSpec((BE, 1), lambda i: (i, 0)),
            pl.BlockSpec((D, DE), lambda i: (0, 0)),
            pl.BlockSpec((1, D), lambda i: (0, 0)),
            pl.BlockSpec((D, 1), lambda i: (0, 0)),
            pl.BlockSpec(memory_space=pltpu.SMEM),
        ],
        out_specs=pl.BlockSpec((BE, 1), lambda i: (i, 0)),
        out_shape=jax.ShapeDtypeStruct((E, 1), jnp.float32),
    )(g, attr, le, w1e, b1, w2, b2)


def kernel(x, edge_index, edge_u, edge_v, edge_attr, log_exposure,
           W1_l, b1_l, W1_r, W2_l, b2_l, W2_r, mW1, mb1, mW2, mb2):
    src = edge_index[0].astype(jnp.int32)
    dst = edge_index[1].astype(jnp.int32)
    u = edge_u.astype(jnp.int32)
    v = edge_v.astype(jnp.int32)
    zkd = jnp.zeros((KS, D), jnp.float32)
    onesd = jnp.ones((KS, D), jnp.float32)

    agg, deg = _agg_deg(x, src, dst, zkd, onesd)
    h1 = _tc_layer1(agg, deg, x, W1_l, b1_l.reshape(1, D), W1_r)
    (aggp2,) = _segsum(h1, src, dst, zkd)
    A, B = _tc_layer2(aggp2, deg, h1, W2_l, b2_l.reshape(1, D), W2_r,
                      mW1[:, :D], mW1[:, D:2 * D])
    (g,) = _gpair(A, B, u, v)
    out = _tc_edge(g, edge_attr, log_exposure.reshape(E, 1),
                   mW1[:, 2 * D:], mb1.reshape(1, D), mW2.reshape(D, 1),
                   mb2)
    return out.reshape(E)


# speedup vs baseline: 5.5936x; 1.1546x over previous
import jax
import jax.numpy as jnp
from jax import lax
from jax.experimental import pallas as pl
from jax.experimental.pallas import tpu as pltpu
from jax.experimental.pallas import tpu_sc as plsc

NC, NS = 2, 16
N, E, D, DE = 10000, 320000, 128, 16
DEGW = 16
EPC = E // NC
EPT = EPC // NS
KS = 80
NCHUNKS = EPT // KS
KG = 200
NCHUNKG = EPT // KG
IB = 25
ZR = 624
ZREM = N - NS * ZR

_mesh = plsc.VectorSubcoreMesh(core_axis_name="c", subcore_axis_name="s",
                               num_cores=NC, num_subcores=NS)


def _row_ranges(s):
    r0 = s * ZR
    return [(r0 + i * KS, min(KS, ZR - i * KS))
            for i in range((ZR + KS - 1) // KS)]


EPT1 = E // NS
NCHUNK1 = EPT1 // KS


def _seg_loop(table, src1d, dst1d, acc, base_e, nblocks,
              sblk, dblk, d0, d1, g0, sem0, g1, sem1):
    def gidx(k):
        return sblk.at[pl.ds(k * KS, KS)]

    def prep(k, dref):
        for i in range(KS // 16):
            dref[pl.ds(i * 16, 16)] = dblk[pl.ds(k * KS + i * 16, 16)]

    def wait_gather(gb, sem):
        pltpu.make_async_copy(table.at[pl.ds(0, KS)], gb, sem).wait()

    def block(ib, carry):
        off = pl.multiple_of(base_e + ib * (IB * KS), 8)
        pltpu.sync_copy(src1d.at[pl.ds(off, IB * KS)], sblk)
        pltpu.sync_copy(dst1d.at[pl.ds(off, IB * KS)], dblk)
        pltpu.async_copy(table.at[gidx(0)], g0, sem0)
        prep(0, d0)

        def pair(jj, cc):
            b = 2 * jj + 1
            c2 = 2 * jj + 2
            pltpu.async_copy(table.at[gidx(b)], g1, sem1)
            prep(b, d1)
            wait_gather(g0, sem0)
            pltpu.sync_copy(g0, acc.at[d0], add=True)

            @pl.when(c2 < IB)
            def _():
                pltpu.async_copy(table.at[gidx(c2)], g0, sem0)
                prep(c2, d0)

            wait_gather(g1, sem1)
            pltpu.sync_copy(g1, acc.at[d1], add=True)
            return cc

        lax.fori_loop(0, IB // 2, pair, 0)
        if IB % 2 == 1:
            wait_gather(g0, sem0)
            pltpu.sync_copy(g0, acc.at[d0], add=True)
        return carry

    lax.fori_loop(0, nblocks, block, 0)


def _zero_acc(zkd, gbuf, acc, s):
    pltpu.sync_copy(zkd, gbuf)
    for off, sz in _row_ranges(s):
        pltpu.sync_copy(gbuf.at[pl.ds(0, sz)], acc.at[pl.ds(off, sz)])

    @pl.when(s == 0)
    def _():
        pltpu.sync_copy(gbuf.at[pl.ds(0, ZREM)], acc.at[pl.ds(NS * ZR, ZREM)])


def _agg_deg_body(table, src1d, dst1d, zkd, onesd, agg_out, deg_out,
                  sblk, dblk, d0, d1, g0, sem0, g1, sem1, acc):
    c = lax.axis_index("c")
    s = lax.axis_index("s")

    _zero_acc(zkd, g0, acc, s)

    @pl.when(c == 1)
    def _():
        pltpu.sync_copy(onesd, g0)

    plsc.subcore_barrier()

    base_e = s * EPT1

    @pl.when(c == 0)
    def _():
        _seg_loop(table, src1d, dst1d, acc, base_e, NCHUNK1 // IB,
                  sblk, dblk, d0, d1, g0, sem0, g1, sem1)

    @pl.when(c == 1)
    def _():
        def block(ib, carry):
            off = pl.multiple_of(base_e + ib * (IB * KS), 8)
            pltpu.sync_copy(dst1d.at[pl.ds(off, IB * KS)], dblk)

            def chunk(k, cc):
                for i in range(KS // 16):
                    d0[pl.ds(i * 16, 16)] = dblk[pl.ds(k * KS + i * 16, 16)]
                pltpu.sync_copy(g0, acc.at[d0], add=True)
                return cc

            lax.fori_loop(0, IB, chunk, 0)
            return carry

        lax.fori_loop(0, NCHUNK1 // IB, block, 0)

    plsc.subcore_barrier()

    r0 = s * ZR

    @pl.when(c == 0)
    def _():
        pltpu.sync_copy(acc.at[pl.ds(r0, ZR)], agg_out.at[pl.ds(r0, ZR)])

    @pl.when(c == 1)
    def _():
        pltpu.sync_copy(acc.at[pl.ds(r0, ZR)], deg_out.at[pl.ds(r0, ZR)])

    @pl.when((s == 0) & (c == 0))
    def _():
        pltpu.sync_copy(acc.at[pl.ds(NS * ZR, ZREM)],
                        agg_out.at[pl.ds(NS * ZR, ZREM)])

    @pl.when((s == 0) & (c == 1))
    def _():
        pltpu.sync_copy(acc.at[pl.ds(NS * ZR, ZREM)],
                        deg_out.at[pl.ds(NS * ZR, ZREM)])


_SEG_SCRATCH = (
    pltpu.VMEM((IB * KS,), jnp.int32),
    pltpu.VMEM((IB * KS,), jnp.int32),
    pltpu.VMEM((KS,), jnp.int32),
    pltpu.VMEM((KS,), jnp.int32),
    pltpu.VMEM((KS, D), jnp.float32),
    pltpu.SemaphoreType.DMA,
    pltpu.VMEM((KS, D), jnp.float32),
    pltpu.SemaphoreType.DMA,
    pltpu.VMEM_SHARED((N, D), jnp.float32),
)

_agg_deg = pl.kernel(
    _agg_deg_body,
    out_type=(jax.ShapeDtypeStruct((N, D), jnp.float32),
              jax.ShapeDtypeStruct((N, D), jnp.float32)),
    mesh=_mesh,
    scratch_types=_SEG_SCRATCH,
)


def _segsum_body(table, src1d, dst1d, zkd, agg_out,
                 sblk, dblk, d0, d1, g0, sem0, g1, sem1, acc):
    c = lax.axis_index("c")
    s = lax.axis_index("s")

    _zero_acc(zkd, g0, acc, s)
    plsc.subcore_barrier()

    base_e = (c * NS + s) * EPT
    _seg_loop(table, src1d, dst1d, acc, base_e, NCHUNKS // IB,
              sblk, dblk, d0, d1, g0, sem0, g1, sem1)

    plsc.subcore_barrier()

    out_base = c * N
    r0 = s * ZR
    pltpu.sync_copy(acc.at[pl.ds(r0, ZR)], agg_out.at[pl.ds(out_base + r0, ZR)])

    @pl.when(s == 0)
    def _():
        pltpu.sync_copy(acc.at[pl.ds(NS * ZR, ZREM)],
                        agg_out.at[pl.ds(out_base + NS * ZR, ZREM)])


_segsum = pl.kernel(
    _segsum_body,
    out_type=(jax.ShapeDtypeStruct((NC * N, D), jnp.float32),),
    mesh=_mesh,
    scratch_types=_SEG_SCRATCH,
)


def _gpair_body(A, B, u1d, v1d, g_out,
                ublk, vblk, ga0, semA0, semB0, ga1, semA1, semB1):
    c = lax.axis_index("c")
    s = lax.axis_index("s")
    base_e = (c * NS + s) * EPT
    pltpu.sync_copy(u1d.at[pl.ds(base_e, EPT)], ublk)
    pltpu.sync_copy(v1d.at[pl.ds(base_e, EPT)], vblk)

    def start_a(j, ga, semA):
        pltpu.async_copy(A.at[ublk.at[pl.ds(j * KG, KG)]], ga, semA)

    def add_b(j, ga, semA, semB):
        pltpu.make_async_copy(A.at[pl.ds(0, KG)], ga, semA).wait()
        pltpu.async_copy(B.at[vblk.at[pl.ds(j * KG, KG)]], ga, semB, add=True)

    def finish(j, ga, semB):
        off = pl.multiple_of(base_e + j * KG, 8)
        pltpu.make_async_copy(B.at[pl.ds(0, KG)], ga, semB).wait()
        pltpu.sync_copy(ga, g_out.at[pl.ds(off, KG)])

    start_a(0, ga0, semA0)

    def pair(jj, carry):
        b = 2 * jj + 1
        cch = 2 * jj + 2
        start_a(b, ga1, semA1)
        add_b(2 * jj, ga0, semA0, semB0)
        finish(2 * jj, ga0, semB0)

        @pl.when(cch < NCHUNKG)
        def _():
            start_a(cch, ga0, semA0)

        add_b(b, ga1, semA1, semB1)
        finish(b, ga1, semB1)
        return carry

    lax.fori_loop(0, NCHUNKG // 2, pair, 0)


_gpair = pl.kernel(
    _gpair_body,
    out_type=(jax.ShapeDtypeStruct((E, D), jnp.float32),),
    mesh=_mesh,
    scratch_types=(
        pltpu.VMEM((EPT,), jnp.int32),
        pltpu.VMEM((EPT,), jnp.int32),
        pltpu.VMEM((KG, D), jnp.float32),
        pltpu.SemaphoreType.DMA,
        pltpu.SemaphoreType.DMA,
        pltpu.VMEM((KG, D), jnp.float32),
        pltpu.SemaphoreType.DMA,
        pltpu.SemaphoreType.DMA,
    ),
)


def _dt(p, w):
    return lax.dot_general(p, w, (((1,), (1,)), ((), ())),
                           preferred_element_type=jnp.float32)


BR = 1000


def _tc1_body(a, dg, xr, wl, bl, wr, out):
    mean = a[...] / jnp.maximum(dg[:, 0:1], 1.0)
    out[...] = jnp.maximum(_dt(mean, wl[...]) + _dt(xr[...], wr[...])
                           + bl[...], 0.0)


def _tc2_body(a0, a1, dg, hr, wl, bl, wr, wu, wv, aout, bout):
    mean = (a0[...] + a1[...]) / jnp.maximum(dg[:, 0:1], 1.0)
    h = jnp.maximum(_dt(mean, wl[...]) + _dt(hr[...], wr[...]) + bl[...], 0.0)
    aout[...] = _dt(h, wu[...])
    bout[...] = _dt(h, wv[...])


def _tc_layer1(agg, deg, x, wl, bl, wr):
    return pl.pallas_call(
        _tc1_body,
        grid=(N // BR,),
        in_specs=[
            pl.BlockSpec((BR, D), lambda i: (i, 0)),
            pl.BlockSpec((BR, D), lambda i: (i, 0)),
            pl.BlockSpec((BR, D), lambda i: (i, 0)),
            pl.BlockSpec((D, D), lambda i: (0, 0)),
            pl.BlockSpec((1, D), lambda i: (0, 0)),
            pl.BlockSpec((D, D), lambda i: (0, 0)),
        ],
        out_specs=pl.BlockSpec((BR, D), lambda i: (i, 0)),
        out_shape=jax.ShapeDtypeStruct((N, D), jnp.float32),
    )(agg, deg, x, wl, bl, wr)


def _tc_layer2(aggp, deg, h1, wl, bl, wr, wu, wv):
    return pl.pallas_call(
        _tc2_body,
        grid=(N // BR,),
        in_specs=[
            pl.BlockSpec((BR, D), lambda i: (i, 0)),
            pl.BlockSpec((BR, D), lambda i: (i + N // BR, 0)),
            pl.BlockSpec((BR, D), lambda i: (i, 0)),
            pl.BlockSpec((BR, D), lambda i: (i, 0)),
            pl.BlockSpec((D, D), lambda i: (0, 0)),
            pl.BlockSpec((1, D), lambda i: (0, 0)),
            pl.BlockSpec((D, D), lambda i: (0, 0)),
            pl.BlockSpec((D, D), lambda i: (0, 0)),
            pl.BlockSpec((D, D), lambda i: (0, 0)),
        ],
        out_specs=[pl.BlockSpec((BR, D), lambda i: (i, 0)),
                   pl.BlockSpec((BR, D), lambda i: (i, 0))],
        out_shape=[jax.ShapeDtypeStruct((N, D), jnp.float32),
                   jax.ShapeDtypeStruct((N, D), jnp.float32)],
    )(aggp, aggp, deg, h1, wl, bl, wr, wu, wv)


BE = 3200


def _tc3_body(g, at, le, w1e, b1, w2, b2, out):
    hmid = jnp.maximum(g[...] + _dt(at[...], w1e[...]) + b1[...], 0.0)
    s = lax.dot_general(hmid, w2[...], (((1,), (0,)), ((), ())),
                        preferred_element_type=jnp.float32)
    out[...] = s + le[...] + b2[0]


def _tc_edge(g, attr, le, w1e, b1, w2, b2):
    return pl.pallas_call(
        _tc3_body,
        grid=(E // BE,),
        in_specs=[
            pl.BlockSpec((BE, D), lambda i: (i, 0)),
            pl.BlockSpec((BE, DE), lambda i: (i, 0)),
            pl.BlockSpec((BE, 1), lambda i: (i, 0)),
            pl.BlockSpec((D, DE), lambda i: (0, 0)),
            pl.BlockSpec((1, D), lambda i: (0, 0)),
            pl.BlockSpec((D, 1), lambda i: (0, 0)),
            pl.BlockSpec(memory_space=pltpu.SMEM),
        ],
        out_specs=pl.BlockSpec((BE, 1), lambda i: (i, 0)),
        out_shape=jax.ShapeDtypeStruct((E, 1), jnp.float32),
    )(g, attr, le, w1e, b1, w2, b2)


def kernel(x, edge_index, edge_u, edge_v, edge_attr, log_exposure,
           W1_l, b1_l, W1_r, W2_l, b2_l, W2_r, mW1, mb1, mW2, mb2):
    src = edge_index[0].astype(jnp.int32)
    dst = edge_index[1].astype(jnp.int32)
    u = edge_u.astype(jnp.int32)
    v = edge_v.astype(jnp.int32)
    zkd = jnp.zeros((KS, D), jnp.float32)
    onesd = jnp.ones((KS, D), jnp.float32)

    agg, deg = _agg_deg(x, src, dst, zkd, onesd)
    h1 = _tc_layer1(agg, deg, x, W1_l, b1_l.reshape(1, D), W1_r)
    (aggp2,) = _segsum(h1, src, dst, zkd)
    A, B = _tc_layer2(aggp2, deg, h1, W2_l, b2_l.reshape(1, D), W2_r,
                      mW1[:, :D], mW1[:, D:2 * D])
    (g,) = _gpair(A, B, u, v)
    out = _tc_edge(g, edge_attr, log_exposure.reshape(E, 1),
                   mW1[:, 2 * D:], mb1.reshape(1, D), mW2.reshape(D, 1),
                   mb2)
    return out.reshape(E)


# speedup vs baseline: 5.6020x; 1.0015x over previous
import jax
import jax.numpy as jnp
from jax import lax
from jax.experimental import pallas as pl
from jax.experimental.pallas import tpu as pltpu
from jax.experimental.pallas import tpu_sc as plsc

NC, NS = 2, 16
N, E, D, DE = 10000, 320000, 128, 16
DEGW = 16
EPC = E // NC
EPT = EPC // NS
KS = 80
NCHUNKS = EPT // KS
KG = 200
NCHUNKG = EPT // KG
IB = 25
ZR = 624
ZREM = N - NS * ZR

_mesh = plsc.VectorSubcoreMesh(core_axis_name="c", subcore_axis_name="s",
                               num_cores=NC, num_subcores=NS)


def _row_ranges(s):
    r0 = s * ZR
    return [(r0 + i * KS, min(KS, ZR - i * KS))
            for i in range((ZR + KS - 1) // KS)]


EPT1 = E // NS
NCHUNK1 = EPT1 // KS


def _seg_loop(table, src1d, dst1d, acc, base_e, nblocks,
              sblk, dblk, d0, d1, g0, sem0, g1, sem1):
    def gidx(k):
        return sblk.at[pl.ds(k * KS, KS)]

    def prep(k, dref):
        for i in range(KS // 16):
            dref[pl.ds(i * 16, 16)] = dblk[pl.ds(k * KS + i * 16, 16)]

    def wait_gather(gb, sem):
        pltpu.make_async_copy(table.at[pl.ds(0, KS)], gb, sem).wait()

    def block(ib, carry):
        off = pl.multiple_of(base_e + ib * (IB * KS), 8)
        pltpu.sync_copy(src1d.at[pl.ds(off, IB * KS)], sblk)
        pltpu.sync_copy(dst1d.at[pl.ds(off, IB * KS)], dblk)
        pltpu.async_copy(table.at[gidx(0)], g0, sem0)
        prep(0, d0)

        def pair(jj, cc):
            b = 2 * jj + 1
            c2 = 2 * jj + 2
            pltpu.async_copy(table.at[gidx(b)], g1, sem1)
            prep(b, d1)
            wait_gather(g0, sem0)
            pltpu.sync_copy(g0, acc.at[d0], add=True)

            @pl.when(c2 < IB)
            def _():
                pltpu.async_copy(table.at[gidx(c2)], g0, sem0)
                prep(c2, d0)

            wait_gather(g1, sem1)
            pltpu.sync_copy(g1, acc.at[d1], add=True)
            return cc

        lax.fori_loop(0, IB // 2, pair, 0)
        if IB % 2 == 1:
            wait_gather(g0, sem0)
            pltpu.sync_copy(g0, acc.at[d0], add=True)
        return carry

    lax.fori_loop(0, nblocks, block, 0)


def _zero_acc(zkd, gbuf, acc, s):
    pltpu.sync_copy(zkd, gbuf)
    for off, sz in _row_ranges(s):
        pltpu.sync_copy(gbuf.at[pl.ds(0, sz)], acc.at[pl.ds(off, sz)])

    @pl.when(s == 0)
    def _():
        pltpu.sync_copy(gbuf.at[pl.ds(0, ZREM)], acc.at[pl.ds(NS * ZR, ZREM)])


def _agg_deg_body(table, src1d, dst1d, zkd, onesd, agg_out, deg_out,
                  sblk, dblk, d0, d1, g0, sem0, g1, sem1, acc):
    c = lax.axis_index("c")
    s = lax.axis_index("s")

    _zero_acc(zkd, g0, acc, s)

    @pl.when(c == 1)
    def _():
        pltpu.sync_copy(onesd, g0)

    plsc.subcore_barrier()

    base_e = s * EPT1

    @pl.when(c == 0)
    def _():
        _seg_loop(table, src1d, dst1d, acc, base_e, NCHUNK1 // IB,
                  sblk, dblk, d0, d1, g0, sem0, g1, sem1)

    @pl.when(c == 1)
    def _():
        def block(ib, carry):
            off = pl.multiple_of(base_e + ib * (IB * KS), 8)
            pltpu.sync_copy(dst1d.at[pl.ds(off, IB * KS)], dblk)

            def chunk(k, cc):
                for i in range(KS // 16):
                    d0[pl.ds(i * 16, 16)] = dblk[pl.ds(k * KS + i * 16, 16)]
                pltpu.sync_copy(g0, acc.at[d0], add=True)
                return cc

            lax.fori_loop(0, IB, chunk, 0)
            return carry

        lax.fori_loop(0, NCHUNK1 // IB, block, 0)

    plsc.subcore_barrier()

    r0 = s * ZR

    @pl.when(c == 0)
    def _():
        pltpu.sync_copy(acc.at[pl.ds(r0, ZR)], agg_out.at[pl.ds(r0, ZR)])

    @pl.when(c == 1)
    def _():
        pltpu.sync_copy(acc.at[pl.ds(r0, ZR)], deg_out.at[pl.ds(r0, ZR)])

    @pl.when((s == 0) & (c == 0))
    def _():
        pltpu.sync_copy(acc.at[pl.ds(NS * ZR, ZREM)],
                        agg_out.at[pl.ds(NS * ZR, ZREM)])

    @pl.when((s == 0) & (c == 1))
    def _():
        pltpu.sync_copy(acc.at[pl.ds(NS * ZR, ZREM)],
                        deg_out.at[pl.ds(NS * ZR, ZREM)])


_SEG_SCRATCH = (
    pltpu.VMEM((IB * KS,), jnp.int32),
    pltpu.VMEM((IB * KS,), jnp.int32),
    pltpu.VMEM((KS,), jnp.int32),
    pltpu.VMEM((KS,), jnp.int32),
    pltpu.VMEM((KS, D), jnp.float32),
    pltpu.SemaphoreType.DMA,
    pltpu.VMEM((KS, D), jnp.float32),
    pltpu.SemaphoreType.DMA,
    pltpu.VMEM_SHARED((N, D), jnp.float32),
)

_agg_deg = pl.kernel(
    _agg_deg_body,
    out_type=(jax.ShapeDtypeStruct((N, D), jnp.float32),
              jax.ShapeDtypeStruct((N, D), jnp.float32)),
    mesh=_mesh,
    scratch_types=_SEG_SCRATCH,
)


def _segsum_body(table, src1d, dst1d, zkd, agg_out,
                 sblk, dblk, d0, d1, g0, sem0, g1, sem1, acc):
    c = lax.axis_index("c")
    s = lax.axis_index("s")

    _zero_acc(zkd, g0, acc, s)
    plsc.subcore_barrier()

    base_e = (c * NS + s) * EPT
    _seg_loop(table, src1d, dst1d, acc, base_e, NCHUNKS // IB,
              sblk, dblk, d0, d1, g0, sem0, g1, sem1)

    plsc.subcore_barrier()

    out_base = c * N
    r0 = s * ZR
    pltpu.sync_copy(acc.at[pl.ds(r0, ZR)], agg_out.at[pl.ds(out_base + r0, ZR)])

    @pl.when(s == 0)
    def _():
        pltpu.sync_copy(acc.at[pl.ds(NS * ZR, ZREM)],
                        agg_out.at[pl.ds(out_base + NS * ZR, ZREM)])


_segsum = pl.kernel(
    _segsum_body,
    out_type=(jax.ShapeDtypeStruct((NC * N, D), jnp.float32),),
    mesh=_mesh,
    scratch_types=_SEG_SCRATCH,
)


def _gpair_body(A, B, u1d, v1d, g_out,
                ublk, vblk, r0, r1, r2, sa0, sa1, sa2, sb0, sb1, sb2):
    c = lax.axis_index("c")
    s = lax.axis_index("s")
    base_e = (c * NS + s) * EPT
    pltpu.sync_copy(u1d.at[pl.ds(base_e, EPT)], ublk)
    pltpu.sync_copy(v1d.at[pl.ds(base_e, EPT)], vblk)

    bufs = (r0, r1, r2)
    sas = (sa0, sa1, sa2)
    sbs = (sb0, sb1, sb2)

    def start_a(j, t):
        pltpu.async_copy(A.at[ublk.at[pl.ds(j * KG, KG)]], bufs[t], sas[t])

    def add_b(j, t):
        pltpu.make_async_copy(A.at[pl.ds(0, KG)], bufs[t], sas[t]).wait()
        pltpu.async_copy(B.at[vblk.at[pl.ds(j * KG, KG)]], bufs[t],
                         sbs[t], add=True)

    def finish(j, t):
        off = pl.multiple_of(base_e + j * KG, 8)
        pltpu.make_async_copy(B.at[pl.ds(0, KG)], bufs[t], sbs[t]).wait()
        pltpu.sync_copy(bufs[t], g_out.at[pl.ds(off, KG)])

    start_a(0, 0)
    start_a(1, 1)
    add_b(0, 0)

    def trio(m, carry):
        k = 3 * m
        for t in range(3):
            start_a(k + t + 2, (t + 2) % 3)
            add_b(k + t + 1, (t + 1) % 3)
            finish(k + t, t)
        return carry

    lax.fori_loop(0, (NCHUNKG - 2) // 3, trio, 0)
    add_b(NCHUNKG - 1, (NCHUNKG - 1) % 3)
    finish(NCHUNKG - 2, (NCHUNKG - 2) % 3)
    finish(NCHUNKG - 1, (NCHUNKG - 1) % 3)


_gpair = pl.kernel(
    _gpair_body,
    out_type=(jax.ShapeDtypeStruct((E, D), jnp.float32),),
    mesh=_mesh,
    scratch_types=(
        pltpu.VMEM((EPT,), jnp.int32),
        pltpu.VMEM((EPT,), jnp.int32),
        pltpu.VMEM((KG, D), jnp.float32),
        pltpu.VMEM((KG, D), jnp.float32),
        pltpu.VMEM((KG, D), jnp.float32),
        pltpu.SemaphoreType.DMA,
        pltpu.SemaphoreType.DMA,
        pltpu.SemaphoreType.DMA,
        pltpu.SemaphoreType.DMA,
        pltpu.SemaphoreType.DMA,
        pltpu.SemaphoreType.DMA,
    ),
)


def _dt(p, w):
    return lax.dot_general(p, w, (((1,), (1,)), ((), ())),
                           preferred_element_type=jnp.float32)


BR = 1000


def _tc1_body(a, dg, xr, wl, bl, wr, out):
    mean = a[...] / jnp.maximum(dg[:, 0:1], 1.0)
    out[...] = jnp.maximum(_dt(mean, wl[...]) + _dt(xr[...], wr[...])
                           + bl[...], 0.0)


def _tc2_body(a0, a1, dg, hr, wl, bl, wr, wu, wv, aout, bout):
    mean = (a0[...] + a1[...]) / jnp.maximum(dg[:, 0:1], 1.0)
    h = jnp.maximum(_dt(mean, wl[...]) + _dt(hr[...], wr[...]) + bl[...], 0.0)
    aout[...] = _dt(h, wu[...])
    bout[...] = _dt(h, wv[...])


def _tc_layer1(agg, deg, x, wl, bl, wr):
    return pl.pallas_call(
        _tc1_body,
        grid=(N // BR,),
        in_specs=[
            pl.BlockSpec((BR, D), lambda i: (i, 0)),
            pl.BlockSpec((BR, D), lambda i: (i, 0)),
            pl.BlockSpec((BR, D), lambda i: (i, 0)),
            pl.BlockSpec((D, D), lambda i: (0, 0)),
            pl.BlockSpec((1, D), lambda i: (0, 0)),
            pl.BlockSpec((D, D), lambda i: (0, 0)),
        ],
        out_specs=pl.BlockSpec((BR, D), lambda i: (i, 0)),
        out_shape=jax.ShapeDtypeStruct((N, D), jnp.float32),
    )(agg, deg, x, wl, bl, wr)


def _tc_layer2(aggp, deg, h1, wl, bl, wr, wu, wv):
    return pl.pallas_call(
        _tc2_body,
        grid=(N // BR,),
        in_specs=[
            pl.BlockSpec((BR, D), lambda i: (i, 0)),
            pl.BlockSpec((BR, D), lambda i: (i + N // BR, 0)),
            pl.BlockSpec((BR, D), lambda i: (i, 0)),
            pl.BlockSpec((BR, D), lambda i: (i, 0)),
            pl.BlockSpec((D, D), lambda i: (0, 0)),
            pl.BlockSpec((1, D), lambda i: (0, 0)),
            pl.BlockSpec((D, D), lambda i: (0, 0)),
            pl.BlockSpec((D, D), lambda i: (0, 0)),
            pl.BlockSpec((D, D), lambda i: (0, 1)),
        ],
        out_specs=[pl.BlockSpec((BR, D), lambda i: (i, 0)),
                   pl.BlockSpec((BR, D), lambda i: (i, 0))],
        out_shape=[jax.ShapeDtypeStruct((N, D), jnp.float32),
                   jax.ShapeDtypeStruct((N, D), jnp.float32)],
    )(aggp, aggp, deg, h1, wl, bl, wr, wu, wv)


BE = 3200


def _tc3_body(g, at, le, w1e, b1, w2, b2, out):
    hmid = jnp.maximum(g[...] + _dt(at[...], w1e[...]) + b1[...], 0.0)
    s = lax.dot_general(hmid, w2[...], (((1,), (0,)), ((), ())),
                        preferred_element_type=jnp.float32)
    out[...] = s + le[...] + b2[0]


def _tc_edge(g, attr, le, w1e, b1, w2, b2):
    return pl.pallas_call(
        _tc3_body,
        grid=(E // BE,),
        in_specs=[
            pl.BlockSpec((BE, D), lambda i: (i, 0)),
            pl.BlockSpec((BE, DE), lambda i: (i, 0)),
            pl.BlockSpec((BE, 1), lambda i: (i, 0)),
            pl.BlockSpec((D, DE), lambda i: (0, 0)),
            pl.BlockSpec((1, D), lambda i: (0, 0)),
            pl.BlockSpec((D, 1), lambda i: (0, 0)),
            pl.BlockSpec(memory_space=pltpu.SMEM),
        ],
        out_specs=pl.BlockSpec((BE, 1), lambda i: (i, 0)),
        out_shape=jax.ShapeDtypeStruct((E, 1), jnp.float32),
    )(g, attr, le, w1e, b1, w2, b2)


def kernel(x, edge_index, edge_u, edge_v, edge_attr, log_exposure,
           W1_l, b1_l, W1_r, W2_l, b2_l, W2_r, mW1, mb1, mW2, mb2):
    src = edge_index[0].astype(jnp.int32)
    dst = edge_index[1].astype(jnp.int32)
    u = edge_u.astype(jnp.int32)
    v = edge_v.astype(jnp.int32)
    zkd = jnp.zeros((KS, D), jnp.float32)
    onesd = jnp.ones((KS, D), jnp.float32)

    agg, deg = _agg_deg(x, src, dst, zkd, onesd)
    h1 = _tc_layer1(agg, deg, x, W1_l, b1_l.reshape(1, D), W1_r)
    (aggp2,) = _segsum(h1, src, dst, zkd)
    A, B = _tc_layer2(aggp2, deg, h1, W2_l, b2_l.reshape(1, D), W2_r,
                      mW1, mW1)
    (g,) = _gpair(A, B, u, v)
    out = _tc_edge(g, edge_attr, log_exposure.reshape(E, 1),
                   mW1[:, 2 * D:], mb1.reshape(1, D), mW2.reshape(D, 1),
                   mb2)
    return out.reshape(E)


# speedup vs baseline: 6.8271x; 1.2187x over previous
import jax
import jax.numpy as jnp
from jax import lax
from jax.experimental import pallas as pl
from jax.experimental.pallas import tpu as pltpu
from jax.experimental.pallas import tpu_sc as plsc

NC, NS = 2, 16
N, E, D, DE = 10000, 320000, 128, 16
DEGW = 16
EPC = E // NC
EPT = EPC // NS
KS = 80
NCHUNKS = EPT // KS
KG = 200
NCHUNKG = EPT // KG
IB = 25
ZR = 624
ZREM = N - NS * ZR

_mesh = plsc.VectorSubcoreMesh(core_axis_name="c", subcore_axis_name="s",
                               num_cores=NC, num_subcores=NS)


def _row_ranges(s):
    r0 = s * ZR
    return [(r0 + i * KS, min(KS, ZR - i * KS))
            for i in range((ZR + KS - 1) // KS)]


EPT1 = E // NS
NCHUNK1 = EPT1 // KS


def _seg_loop(table, src1d, dst1d, acc, base_e, nblocks,
              sblk, dblk, d0, d1, g0, sem0, g1, sem1):
    def gidx(k):
        return sblk.at[pl.ds(k * KS, KS)]

    def prep(k, dref):
        for i in range(KS // 16):
            dref[pl.ds(i * 16, 16)] = dblk[pl.ds(k * KS + i * 16, 16)]

    def wait_gather(gb, sem):
        pltpu.make_async_copy(table.at[pl.ds(0, KS)], gb, sem).wait()

    def block(ib, carry):
        off = pl.multiple_of(base_e + ib * (IB * KS), 8)
        pltpu.sync_copy(src1d.at[pl.ds(off, IB * KS)], sblk)
        pltpu.sync_copy(dst1d.at[pl.ds(off, IB * KS)], dblk)
        pltpu.async_copy(table.at[gidx(0)], g0, sem0)
        prep(0, d0)

        def pair(jj, cc):
            b = 2 * jj + 1
            c2 = 2 * jj + 2
            pltpu.async_copy(table.at[gidx(b)], g1, sem1)
            prep(b, d1)
            wait_gather(g0, sem0)
            pltpu.sync_copy(g0, acc.at[d0], add=True)

            @pl.when(c2 < IB)
            def _():
                pltpu.async_copy(table.at[gidx(c2)], g0, sem0)
                prep(c2, d0)

            wait_gather(g1, sem1)
            pltpu.sync_copy(g1, acc.at[d1], add=True)
            return cc

        lax.fori_loop(0, IB // 2, pair, 0)
        if IB % 2 == 1:
            wait_gather(g0, sem0)
            pltpu.sync_copy(g0, acc.at[d0], add=True)
        return carry

    lax.fori_loop(0, nblocks, block, 0)


def _zero_acc(zkd, gbuf, acc, s):
    pltpu.sync_copy(zkd, gbuf)
    for off, sz in _row_ranges(s):
        pltpu.sync_copy(gbuf.at[pl.ds(0, sz)], acc.at[pl.ds(off, sz)])

    @pl.when(s == 0)
    def _():
        pltpu.sync_copy(gbuf.at[pl.ds(0, ZREM)], acc.at[pl.ds(NS * ZR, ZREM)])


def _agg_deg_body(table, src1d, dst1d, zkd, onesd, agg_out, deg_out,
                  sblk, dblk, d0, d1, g0, sem0, g1, sem1, acc):
    c = lax.axis_index("c")
    s = lax.axis_index("s")

    _zero_acc(zkd, g0, acc, s)

    @pl.when(c == 1)
    def _():
        pltpu.sync_copy(onesd, g0)

    plsc.subcore_barrier()

    base_e = s * EPT1

    @pl.when(c == 0)
    def _():
        _seg_loop(table, src1d, dst1d, acc, base_e, NCHUNK1 // IB,
                  sblk, dblk, d0, d1, g0, sem0, g1, sem1)

    @pl.when(c == 1)
    def _():
        def block(ib, carry):
            off = pl.multiple_of(base_e + ib * (IB * KS), 8)
            pltpu.sync_copy(dst1d.at[pl.ds(off, IB * KS)], dblk)

            def chunk(k, cc):
                for i in range(KS // 16):
                    d0[pl.ds(i * 16, 16)] = dblk[pl.ds(k * KS + i * 16, 16)]
                pltpu.sync_copy(g0, acc.at[d0], add=True)
                return cc

            lax.fori_loop(0, IB, chunk, 0)
            return carry

        lax.fori_loop(0, NCHUNK1 // IB, block, 0)

    plsc.subcore_barrier()

    r0 = s * ZR

    @pl.when(c == 0)
    def _():
        pltpu.sync_copy(acc.at[pl.ds(r0, ZR)], agg_out.at[pl.ds(r0, ZR)])

    @pl.when(c == 1)
    def _():
        pltpu.sync_copy(acc.at[pl.ds(r0, ZR)], deg_out.at[pl.ds(r0, ZR)])

    @pl.when((s == 0) & (c == 0))
    def _():
        pltpu.sync_copy(acc.at[pl.ds(NS * ZR, ZREM)],
                        agg_out.at[pl.ds(NS * ZR, ZREM)])

    @pl.when((s == 0) & (c == 1))
    def _():
        pltpu.sync_copy(acc.at[pl.ds(NS * ZR, ZREM)],
                        deg_out.at[pl.ds(NS * ZR, ZREM)])


_SEG_SCRATCH = (
    pltpu.VMEM((IB * KS,), jnp.int32),
    pltpu.VMEM((IB * KS,), jnp.int32),
    pltpu.VMEM((KS,), jnp.int32),
    pltpu.VMEM((KS,), jnp.int32),
    pltpu.VMEM((KS, D), jnp.float32),
    pltpu.SemaphoreType.DMA,
    pltpu.VMEM((KS, D), jnp.float32),
    pltpu.SemaphoreType.DMA,
    pltpu.VMEM_SHARED((N, D), jnp.float32),
)

_agg_deg = pl.kernel(
    _agg_deg_body,
    out_type=(jax.ShapeDtypeStruct((N, D), jnp.float32),
              jax.ShapeDtypeStruct((N, D), jnp.float32)),
    mesh=_mesh,
    scratch_types=_SEG_SCRATCH,
)


def _segsum_body(table, src1d, dst1d, zkd, agg_out,
                 sblk, dblk, d0, d1, g0, sem0, g1, sem1, acc):
    c = lax.axis_index("c")
    s = lax.axis_index("s")

    _zero_acc(zkd, g0, acc, s)
    plsc.subcore_barrier()

    base_e = (c * NS + s) * EPT
    _seg_loop(table, src1d, dst1d, acc, base_e, NCHUNKS // IB,
              sblk, dblk, d0, d1, g0, sem0, g1, sem1)

    plsc.subcore_barrier()

    out_base = c * N
    r0 = s * ZR
    pltpu.sync_copy(acc.at[pl.ds(r0, ZR)], agg_out.at[pl.ds(out_base + r0, ZR)])

    @pl.when(s == 0)
    def _():
        pltpu.sync_copy(acc.at[pl.ds(NS * ZR, ZREM)],
                        agg_out.at[pl.ds(out_base + NS * ZR, ZREM)])


_segsum = pl.kernel(
    _segsum_body,
    out_type=(jax.ShapeDtypeStruct((NC * N, D), jnp.float32),),
    mesh=_mesh,
    scratch_types=_SEG_SCRATCH,
)


def _gpair_body(A, B, u1d, v1d, g_out,
                ublk, vblk, r0, r1, r2, sa0, sa1, sa2, sb0, sb1, sb2):
    c = lax.axis_index("c")
    s = lax.axis_index("s")
    base_e = (c * NS + s) * EPT
    pltpu.sync_copy(u1d.at[pl.ds(base_e, EPT)], ublk)
    pltpu.sync_copy(v1d.at[pl.ds(base_e, EPT)], vblk)

    bufs = (r0, r1, r2)
    sas = (sa0, sa1, sa2)
    sbs = (sb0, sb1, sb2)

    def start_a(j, t):
        pltpu.async_copy(A.at[ublk.at[pl.ds(j * KG, KG)]], bufs[t], sas[t])

    def add_b(j, t):
        pltpu.make_async_copy(A.at[pl.ds(0, KG)], bufs[t], sas[t]).wait()
        pltpu.async_copy(B.at[vblk.at[pl.ds(j * KG, KG)]], bufs[t],
                         sbs[t], add=True)

    def finish(j, t):
        off = pl.multiple_of(base_e + j * KG, 8)
        pltpu.make_async_copy(B.at[pl.ds(0, KG)], bufs[t], sbs[t]).wait()
        pltpu.sync_copy(bufs[t], g_out.at[pl.ds(off, KG)])

    start_a(0, 0)
    start_a(1, 1)
    add_b(0, 0)

    def trio(m, carry):
        k = 3 * m
        for t in range(3):
            start_a(k + t + 2, (t + 2) % 3)
            add_b(k + t + 1, (t + 1) % 3)
            finish(k + t, t)
        return carry

    lax.fori_loop(0, (NCHUNKG - 2) // 3, trio, 0)
    add_b(NCHUNKG - 1, (NCHUNKG - 1) % 3)
    finish(NCHUNKG - 2, (NCHUNKG - 2) % 3)
    finish(NCHUNKG - 1, (NCHUNKG - 1) % 3)


_gpair = pl.kernel(
    _gpair_body,
    out_type=(jax.ShapeDtypeStruct((E, D), jnp.float32),),
    mesh=_mesh,
    scratch_types=(
        pltpu.VMEM((EPT,), jnp.int32),
        pltpu.VMEM((EPT,), jnp.int32),
        pltpu.VMEM((KG, D), jnp.float32),
        pltpu.VMEM((KG, D), jnp.float32),
        pltpu.VMEM((KG, D), jnp.float32),
        pltpu.SemaphoreType.DMA,
        pltpu.SemaphoreType.DMA,
        pltpu.SemaphoreType.DMA,
        pltpu.SemaphoreType.DMA,
        pltpu.SemaphoreType.DMA,
        pltpu.SemaphoreType.DMA,
    ),
)


def _dt(p, w):
    return lax.dot_general(p, w, (((1,), (1,)), ((), ())),
                           preferred_element_type=jnp.float32)


BR = 1000


def _tc1_body(a, dg, xr, wl, bl, wr, out):
    mean = a[...] / jnp.maximum(dg[:, 0:1], 1.0)
    out[...] = jnp.maximum(_dt(mean, wl[...]) + _dt(xr[...], wr[...])
                           + bl[...], 0.0)


def _tc2_body(a0, a1, dg, hr, wl, bl, wr, wu, wv, aout, bout):
    mean = (a0[...] + a1[...]) / jnp.maximum(dg[:, 0:1], 1.0)
    h = jnp.maximum(_dt(mean, wl[...]) + _dt(hr[...], wr[...]) + bl[...], 0.0)
    aout[...] = _dt(h, wu[...])
    bout[...] = _dt(h, wv[...])


def _tc_layer1(agg, deg, x, wl, bl, wr):
    return pl.pallas_call(
        _tc1_body,
        grid=(N // BR,),
        in_specs=[
            pl.BlockSpec((BR, D), lambda i: (i, 0)),
            pl.BlockSpec((BR, D), lambda i: (i, 0)),
            pl.BlockSpec((BR, D), lambda i: (i, 0)),
            pl.BlockSpec((D, D), lambda i: (0, 0)),
            pl.BlockSpec((1, D), lambda i: (0, 0)),
            pl.BlockSpec((D, D), lambda i: (0, 0)),
        ],
        out_specs=pl.BlockSpec((BR, D), lambda i: (i, 0)),
        out_shape=jax.ShapeDtypeStruct((N, D), jnp.float32),
    )(agg, deg, x, wl, bl, wr)


def _tc_layer2(aggp, deg, h1, wl, bl, wr, wu, wv):
    return pl.pallas_call(
        _tc2_body,
        grid=(N // BR,),
        in_specs=[
            pl.BlockSpec((BR, D), lambda i: (i, 0)),
            pl.BlockSpec((BR, D), lambda i: (i + N // BR, 0)),
            pl.BlockSpec((BR, D), lambda i: (i, 0)),
            pl.BlockSpec((BR, D), lambda i: (i, 0)),
            pl.BlockSpec((D, D), lambda i: (0, 0)),
            pl.BlockSpec((1, D), lambda i: (0, 0)),
            pl.BlockSpec((D, D), lambda i: (0, 0)),
            pl.BlockSpec((D, D), lambda i: (0, 0)),
            pl.BlockSpec((D, D), lambda i: (0, 1)),
        ],
        out_specs=[pl.BlockSpec((BR, D), lambda i: (i, 0)),
                   pl.BlockSpec((BR, D), lambda i: (i, 0))],
        out_shape=[jax.ShapeDtypeStruct((N, D), jnp.float32),
                   jax.ShapeDtypeStruct((N, D), jnp.float32)],
    )(aggp, aggp, deg, h1, wl, bl, wr, wu, wv)


BE = 3200


RB = BE // 128


def _tc3_body(g, at, le, w1e, b1, w2, b2, out):
    hmid = jnp.maximum(g[...] + _dt(at[...], w1e[...]) + b1[...], 0.0)
    h3 = hmid.reshape(RB, 128, D)
    s = lax.dot_general(h3, w2[...], (((2,), (0,)), ((), ())),
                        preferred_element_type=jnp.float32)
    out[...] = (s + le[0] + b2[0]).reshape(1, RB, 128)


def _tc_edge(g, attr, le, w1e, b1, w2, b2):
    return pl.pallas_call(
        _tc3_body,
        grid=(E // BE,),
        in_specs=[
            pl.BlockSpec((BE, D), lambda i: (i, 0)),
            pl.BlockSpec((BE, DE), lambda i: (i, 0)),
            pl.BlockSpec((1, RB, 128), lambda i: (i, 0, 0)),
            pl.BlockSpec((D, DE), lambda i: (0, 0)),
            pl.BlockSpec((1, D), lambda i: (0, 0)),
            pl.BlockSpec((D,), lambda i: (0,)),
            pl.BlockSpec(memory_space=pltpu.SMEM),
        ],
        out_specs=pl.BlockSpec((1, RB, 128), lambda i: (i, 0, 0)),
        out_shape=jax.ShapeDtypeStruct((E // BE, RB, 128), jnp.float32),
    )(g, attr, le, w1e, b1, w2, b2)


def kernel(x, edge_index, edge_u, edge_v, edge_attr, log_exposure,
           W1_l, b1_l, W1_r, W2_l, b2_l, W2_r, mW1, mb1, mW2, mb2):
    src = edge_index[0].astype(jnp.int32)
    dst = edge_index[1].astype(jnp.int32)
    u = edge_u.astype(jnp.int32)
    v = edge_v.astype(jnp.int32)
    zkd = jnp.zeros((KS, D), jnp.float32)
    onesd = jnp.ones((KS, D), jnp.float32)

    agg, deg = _agg_deg(x, src, dst, zkd, onesd)
    h1 = _tc_layer1(agg, deg, x, W1_l, b1_l.reshape(1, D), W1_r)
    (aggp2,) = _segsum(h1, src, dst, zkd)
    A, B = _tc_layer2(aggp2, deg, h1, W2_l, b2_l.reshape(1, D), W2_r,
                      mW1, mW1)
    (g,) = _gpair(A, B, u, v)
    out = _tc_edge(g, edge_attr, log_exposure.reshape(E // BE, RB, 128),
                   mW1[:, 2 * D:], mb1.reshape(1, D), mW2.reshape(D),
                   mb2)
    return out.reshape(E)


# speedup vs baseline: 7.1694x; 1.0501x over previous
import jax
import jax.numpy as jnp
from jax import lax
from jax.experimental import pallas as pl
from jax.experimental.pallas import tpu as pltpu
from jax.experimental.pallas import tpu_sc as plsc

NC, NS = 2, 16
N, E, D, DE = 10000, 320000, 128, 16
DEGW = 16
EPC = E // NC
EPT = EPC // NS
KS = 80
NCHUNKS = EPT // KS
KG = 200
NCHUNKG = EPT // KG
IB = 25
ZR = 624
ZREM = N - NS * ZR

_mesh = plsc.VectorSubcoreMesh(core_axis_name="c", subcore_axis_name="s",
                               num_cores=NC, num_subcores=NS)


def _row_ranges(s):
    r0 = s * ZR
    return [(r0 + i * KS, min(KS, ZR - i * KS))
            for i in range((ZR + KS - 1) // KS)]


EPT1 = E // NS
NCHUNK1 = EPT1 // KS


def _seg_loop(table, src1d, dst1d, acc, base_e, nblocks,
              sblk, dblk, d0, d1, g0, sem0, g1, sem1):
    def gidx(k):
        return sblk.at[pl.ds(k * KS, KS)]

    def prep(k, dref):
        for i in range(KS // 16):
            dref[pl.ds(i * 16, 16)] = dblk[pl.ds(k * KS + i * 16, 16)]

    def wait_gather(gb, sem):
        pltpu.make_async_copy(table.at[pl.ds(0, KS)], gb, sem).wait()

    def block(ib, carry):
        off = pl.multiple_of(base_e + ib * (IB * KS), 8)
        pltpu.sync_copy(src1d.at[pl.ds(off, IB * KS)], sblk)
        pltpu.sync_copy(dst1d.at[pl.ds(off, IB * KS)], dblk)
        pltpu.async_copy(table.at[gidx(0)], g0, sem0)
        prep(0, d0)

        def pair(jj, cc):
            b = 2 * jj + 1
            c2 = 2 * jj + 2
            pltpu.async_copy(table.at[gidx(b)], g1, sem1)
            prep(b, d1)
            wait_gather(g0, sem0)
            pltpu.sync_copy(g0, acc.at[d0], add=True)

            @pl.when(c2 < IB)
            def _():
                pltpu.async_copy(table.at[gidx(c2)], g0, sem0)
                prep(c2, d0)

            wait_gather(g1, sem1)
            pltpu.sync_copy(g1, acc.at[d1], add=True)
            return cc

        lax.fori_loop(0, IB // 2, pair, 0)
        if IB % 2 == 1:
            wait_gather(g0, sem0)
            pltpu.sync_copy(g0, acc.at[d0], add=True)
        return carry

    lax.fori_loop(0, nblocks, block, 0)


def _zero_acc(zkd, gbuf, acc, s):
    pltpu.sync_copy(zkd, gbuf)
    for off, sz in _row_ranges(s):
        pltpu.sync_copy(gbuf.at[pl.ds(0, sz)], acc.at[pl.ds(off, sz)])

    @pl.when(s == 0)
    def _():
        pltpu.sync_copy(gbuf.at[pl.ds(0, ZREM)], acc.at[pl.ds(NS * ZR, ZREM)])


def _agg_deg_body(table, src1d, dst1d, zkd, onesd, agg_out, deg_out,
                  sblk, dblk, d0, d1, g0, sem0, g1, sem1, acc):
    c = lax.axis_index("c")
    s = lax.axis_index("s")

    _zero_acc(zkd, g0, acc, s)

    @pl.when(c == 1)
    def _():
        pltpu.sync_copy(onesd, g0)

    plsc.subcore_barrier()

    base_e = s * EPT1

    @pl.when(c == 0)
    def _():
        _seg_loop(table, src1d, dst1d, acc, base_e, NCHUNK1 // IB,
                  sblk, dblk, d0, d1, g0, sem0, g1, sem1)

    @pl.when(c == 1)
    def _():
        def block(ib, carry):
            off = pl.multiple_of(base_e + ib * (IB * KS), 8)
            pltpu.sync_copy(dst1d.at[pl.ds(off, IB * KS)], dblk)

            def chunk(k, cc):
                for i in range(KS // 16):
                    d0[pl.ds(i * 16, 16)] = dblk[pl.ds(k * KS + i * 16, 16)]
                pltpu.sync_copy(g0, acc.at[d0], add=True)
                return cc

            lax.fori_loop(0, IB, chunk, 0)
            return carry

        lax.fori_loop(0, NCHUNK1 // IB, block, 0)

    plsc.subcore_barrier()

    r0 = s * ZR

    @pl.when(c == 0)
    def _():
        pltpu.sync_copy(acc.at[pl.ds(r0, ZR)], agg_out.at[pl.ds(r0, ZR)])

    @pl.when(c == 1)
    def _():
        pltpu.sync_copy(acc.at[pl.ds(r0, ZR)], deg_out.at[pl.ds(r0, ZR)])

    @pl.when((s == 0) & (c == 0))
    def _():
        pltpu.sync_copy(acc.at[pl.ds(NS * ZR, ZREM)],
                        agg_out.at[pl.ds(NS * ZR, ZREM)])

    @pl.when((s == 0) & (c == 1))
    def _():
        pltpu.sync_copy(acc.at[pl.ds(NS * ZR, ZREM)],
                        deg_out.at[pl.ds(NS * ZR, ZREM)])


_SEG_SCRATCH = (
    pltpu.VMEM((IB * KS,), jnp.int32),
    pltpu.VMEM((IB * KS,), jnp.int32),
    pltpu.VMEM((KS,), jnp.int32),
    pltpu.VMEM((KS,), jnp.int32),
    pltpu.VMEM((KS, D), jnp.float32),
    pltpu.SemaphoreType.DMA,
    pltpu.VMEM((KS, D), jnp.float32),
    pltpu.SemaphoreType.DMA,
    pltpu.VMEM_SHARED((N, D), jnp.float32),
)

_agg_deg = pl.kernel(
    _agg_deg_body,
    out_type=(jax.ShapeDtypeStruct((N, D), jnp.float32),
              jax.ShapeDtypeStruct((N, D), jnp.float32)),
    mesh=_mesh,
    scratch_types=_SEG_SCRATCH,
)


def _segsum_body(table, src1d, dst1d, zkd, agg_out,
                 sblk, dblk, d0, d1, g0, sem0, g1, sem1, acc):
    c = lax.axis_index("c")
    s = lax.axis_index("s")

    _zero_acc(zkd, g0, acc, s)
    plsc.subcore_barrier()

    base_e = (c * NS + s) * EPT
    _seg_loop(table, src1d, dst1d, acc, base_e, NCHUNKS // IB,
              sblk, dblk, d0, d1, g0, sem0, g1, sem1)

    plsc.subcore_barrier()

    out_base = c * N
    r0 = s * ZR
    pltpu.sync_copy(acc.at[pl.ds(r0, ZR)], agg_out.at[pl.ds(out_base + r0, ZR)])

    @pl.when(s == 0)
    def _():
        pltpu.sync_copy(acc.at[pl.ds(NS * ZR, ZREM)],
                        agg_out.at[pl.ds(out_base + NS * ZR, ZREM)])


_segsum = pl.kernel(
    _segsum_body,
    out_type=(jax.ShapeDtypeStruct((NC * N, D), jnp.float32),),
    mesh=_mesh,
    scratch_types=_SEG_SCRATCH,
)


def _gpair_body(A, B, u1d, v1d, g_out,
                ublk, vblk, r0, r1, r2, sa0, sa1, sa2, sb0, sb1, sb2):
    c = lax.axis_index("c")
    s = lax.axis_index("s")
    base_e = (c * NS + s) * EPT
    pltpu.sync_copy(u1d.at[pl.ds(base_e, EPT)], ublk)
    pltpu.sync_copy(v1d.at[pl.ds(base_e, EPT)], vblk)

    bufs = (r0, r1, r2)
    sas = (sa0, sa1, sa2)
    sbs = (sb0, sb1, sb2)

    def start_a(j, t):
        pltpu.async_copy(A.at[ublk.at[pl.ds(j * KG, KG)]], bufs[t], sas[t])

    def add_b(j, t):
        pltpu.make_async_copy(A.at[pl.ds(0, KG)], bufs[t], sas[t]).wait()
        pltpu.async_copy(B.at[vblk.at[pl.ds(j * KG, KG)]], bufs[t],
                         sbs[t], add=True)

    def finish(j, t):
        off = pl.multiple_of(base_e + j * KG, 8)
        pltpu.make_async_copy(B.at[pl.ds(0, KG)], bufs[t], sbs[t]).wait()
        pltpu.sync_copy(bufs[t], g_out.at[pl.ds(off, KG)])

    start_a(0, 0)
    start_a(1, 1)
    add_b(0, 0)

    def trio(m, carry):
        k = 3 * m
        for t in range(3):
            start_a(k + t + 2, (t + 2) % 3)
            add_b(k + t + 1, (t + 1) % 3)
            finish(k + t, t)
        return carry

    lax.fori_loop(0, (NCHUNKG - 2) // 3, trio, 0)
    add_b(NCHUNKG - 1, (NCHUNKG - 1) % 3)
    finish(NCHUNKG - 2, (NCHUNKG - 2) % 3)
    finish(NCHUNKG - 1, (NCHUNKG - 1) % 3)


_gpair = pl.kernel(
    _gpair_body,
    out_type=(jax.ShapeDtypeStruct((E, D), jnp.float32),),
    mesh=_mesh,
    scratch_types=(
        pltpu.VMEM((EPT,), jnp.int32),
        pltpu.VMEM((EPT,), jnp.int32),
        pltpu.VMEM((KG, D), jnp.float32),
        pltpu.VMEM((KG, D), jnp.float32),
        pltpu.VMEM((KG, D), jnp.float32),
        pltpu.SemaphoreType.DMA,
        pltpu.SemaphoreType.DMA,
        pltpu.SemaphoreType.DMA,
        pltpu.SemaphoreType.DMA,
        pltpu.SemaphoreType.DMA,
        pltpu.SemaphoreType.DMA,
    ),
)


def _dt(p, w):
    return lax.dot_general(p, w, (((1,), (1,)), ((), ())),
                           preferred_element_type=jnp.float32)


BR = 1000


def _tc1_body(a, dg, xr, wl, bl, wr, out):
    mean = a[...] / jnp.maximum(dg[:, 0:1], 1.0)
    out[...] = jnp.maximum(_dt(mean, wl[...]) + _dt(xr[...], wr[...])
                           + bl[...], 0.0)


def _tc2_body(a0, a1, dg, hr, wl, bl, wr, wu, wv, aout, bout):
    mean = (a0[...] + a1[...]) / jnp.maximum(dg[:, 0:1], 1.0)
    h = jnp.maximum(_dt(mean, wl[...]) + _dt(hr[...], wr[...]) + bl[...], 0.0)
    aout[...] = _dt(h, wu[...])
    bout[...] = _dt(h, wv[...])


def _tc_layer1(agg, deg, x, wl, bl, wr):
    return pl.pallas_call(
        _tc1_body,
        grid=(N // BR,),
        in_specs=[
            pl.BlockSpec((BR, D), lambda i: (i, 0)),
            pl.BlockSpec((BR, D), lambda i: (i, 0)),
            pl.BlockSpec((BR, D), lambda i: (i, 0)),
            pl.BlockSpec((D, D), lambda i: (0, 0)),
            pl.BlockSpec((1, D), lambda i: (0, 0)),
            pl.BlockSpec((D, D), lambda i: (0, 0)),
        ],
        out_specs=pl.BlockSpec((BR, D), lambda i: (i, 0)),
        out_shape=jax.ShapeDtypeStruct((N, D), jnp.float32),
    )(agg, deg, x, wl, bl, wr)


def _tc_layer2(aggp, deg, h1, wl, bl, wr, wu, wv):
    return pl.pallas_call(
        _tc2_body,
        grid=(N // BR,),
        in_specs=[
            pl.BlockSpec((BR, D), lambda i: (i, 0)),
            pl.BlockSpec((BR, D), lambda i: (i + N // BR, 0)),
            pl.BlockSpec((BR, D), lambda i: (i, 0)),
            pl.BlockSpec((BR, D), lambda i: (i, 0)),
            pl.BlockSpec((D, D), lambda i: (0, 0)),
            pl.BlockSpec((1, D), lambda i: (0, 0)),
            pl.BlockSpec((D, D), lambda i: (0, 0)),
            pl.BlockSpec((D, D), lambda i: (0, 0)),
            pl.BlockSpec((D, D), lambda i: (0, 1)),
        ],
        out_specs=[pl.BlockSpec((BR, D), lambda i: (i, 0)),
                   pl.BlockSpec((BR, D), lambda i: (i, 0))],
        out_shape=[jax.ShapeDtypeStruct((N, D), jnp.float32),
                   jax.ShapeDtypeStruct((N, D), jnp.float32)],
    )(aggp, aggp, deg, h1, wl, bl, wr, wu, wv)


BE = 12800


RB = BE // 128


def _tc3_body(g, at, le, w1e, b1, w2, b2, out):
    hmid = jnp.maximum(g[...] + _dt(at[...], w1e[...]) + b1[...], 0.0)
    h3 = hmid.reshape(RB, 128, D)
    s = lax.dot_general(h3, w2[...], (((2,), (0,)), ((), ())),
                        preferred_element_type=jnp.float32)
    out[...] = (s + le[0] + b2[0]).reshape(1, RB, 128)


def _tc_edge(g, attr, le, w1e, b1, w2, b2):
    return pl.pallas_call(
        _tc3_body,
        grid=(E // BE,),
        in_specs=[
            pl.BlockSpec((BE, D), lambda i: (i, 0)),
            pl.BlockSpec((BE, DE), lambda i: (i, 0)),
            pl.BlockSpec((1, RB, 128), lambda i: (i, 0, 0)),
            pl.BlockSpec((D, DE), lambda i: (0, 0)),
            pl.BlockSpec((1, D), lambda i: (0, 0)),
            pl.BlockSpec((D,), lambda i: (0,)),
            pl.BlockSpec(memory_space=pltpu.SMEM),
        ],
        out_specs=pl.BlockSpec((1, RB, 128), lambda i: (i, 0, 0)),
        out_shape=jax.ShapeDtypeStruct((E // BE, RB, 128), jnp.float32),
    )(g, attr, le, w1e, b1, w2, b2)


def kernel(x, edge_index, edge_u, edge_v, edge_attr, log_exposure,
           W1_l, b1_l, W1_r, W2_l, b2_l, W2_r, mW1, mb1, mW2, mb2):
    src = edge_index[0].astype(jnp.int32)
    dst = edge_index[1].astype(jnp.int32)
    u = edge_u.astype(jnp.int32)
    v = edge_v.astype(jnp.int32)
    zkd = jnp.zeros((KS, D), jnp.float32)
    onesd = jnp.ones((KS, D), jnp.float32)

    agg, deg = _agg_deg(x, src, dst, zkd, onesd)
    h1 = _tc_layer1(agg, deg, x, W1_l, b1_l.reshape(1, D), W1_r)
    (aggp2,) = _segsum(h1, src, dst, zkd)
    A, B = _tc_layer2(aggp2, deg, h1, W2_l, b2_l.reshape(1, D), W2_r,
                      mW1, mW1)
    (g,) = _gpair(A, B, u, v)
    out = _tc_edge(g, edge_attr, log_exposure.reshape(E // BE, RB, 128),
                   mW1[:, 2 * D:], mb1.reshape(1, D), mW2.reshape(D),
                   mb2)
    return out.reshape(E)


# speedup vs baseline: 7.1885x; 1.0027x over previous
import jax
import jax.numpy as jnp
from jax import lax
from jax.experimental import pallas as pl
from jax.experimental.pallas import tpu as pltpu
from jax.experimental.pallas import tpu_sc as plsc

NC, NS = 2, 16
N, E, D, DE = 10000, 320000, 128, 16
DEGW = 16
EPC = E // NC
EPT = EPC // NS
KS = 80
NCHUNKS = EPT // KS
KG = 200
NCHUNKG = EPT // KG
IB = 25
ZR = 624
ZREM = N - NS * ZR

_mesh = plsc.VectorSubcoreMesh(core_axis_name="c", subcore_axis_name="s",
                               num_cores=NC, num_subcores=NS)


def _row_ranges(s):
    r0 = s * ZR
    return [(r0 + i * KS, min(KS, ZR - i * KS))
            for i in range((ZR + KS - 1) // KS)]


EPT1 = E // NS
NCHUNK1 = EPT1 // KS


def _seg_loop(table, src1d, dst1d, acc, base_e, nblocks,
              sblk, dblk, d0, d1, g0, sem0, g1, sem1):
    def gidx(k):
        return sblk.at[pl.ds(k * KS, KS)]

    def prep(k, dref):
        for i in range(KS // 16):
            dref[pl.ds(i * 16, 16)] = dblk[pl.ds(k * KS + i * 16, 16)]

    def wait_gather(gb, sem):
        pltpu.make_async_copy(table.at[pl.ds(0, KS)], gb, sem).wait()

    def block(ib, carry):
        off = pl.multiple_of(base_e + ib * (IB * KS), 8)
        pltpu.sync_copy(src1d.at[pl.ds(off, IB * KS)], sblk)
        pltpu.sync_copy(dst1d.at[pl.ds(off, IB * KS)], dblk)
        pltpu.async_copy(table.at[gidx(0)], g0, sem0)
        prep(0, d0)

        def pair(jj, cc):
            b = 2 * jj + 1
            c2 = 2 * jj + 2
            pltpu.async_copy(table.at[gidx(b)], g1, sem1)
            prep(b, d1)
            wait_gather(g0, sem0)
            pltpu.sync_copy(g0, acc.at[d0], add=True)

            @pl.when(c2 < IB)
            def _():
                pltpu.async_copy(table.at[gidx(c2)], g0, sem0)
                prep(c2, d0)

            wait_gather(g1, sem1)
            pltpu.sync_copy(g1, acc.at[d1], add=True)
            return cc

        lax.fori_loop(0, IB // 2, pair, 0)
        if IB % 2 == 1:
            wait_gather(g0, sem0)
            pltpu.sync_copy(g0, acc.at[d0], add=True)
        return carry

    lax.fori_loop(0, nblocks, block, 0)


def _zero_acc(zkd, gbuf, acc, s):
    pltpu.sync_copy(zkd, gbuf)
    for off, sz in _row_ranges(s):
        pltpu.sync_copy(gbuf.at[pl.ds(0, sz)], acc.at[pl.ds(off, sz)])

    @pl.when(s == 0)
    def _():
        pltpu.sync_copy(gbuf.at[pl.ds(0, ZREM)], acc.at[pl.ds(NS * ZR, ZREM)])


def _agg_deg_body(table, src1d, dst1d, zkd, onesd, agg_out, deg_out,
                  sblk, dblk, d0, d1, g0, sem0, g1, sem1, acc):
    c = lax.axis_index("c")
    s = lax.axis_index("s")

    _zero_acc(zkd, g0, acc, s)

    @pl.when(c == 1)
    def _():
        pltpu.sync_copy(onesd, g0)

    plsc.subcore_barrier()

    base_e = s * EPT1

    @pl.when(c == 0)
    def _():
        _seg_loop(table, src1d, dst1d, acc, base_e, NCHUNK1 // IB,
                  sblk, dblk, d0, d1, g0, sem0, g1, sem1)

    @pl.when(c == 1)
    def _():
        def block(ib, carry):
            off = pl.multiple_of(base_e + ib * (IB * KS), 8)
            pltpu.sync_copy(dst1d.at[pl.ds(off, IB * KS)], dblk)

            def chunk(k, cc):
                for i in range(KS // 16):
                    d0[pl.ds(i * 16, 16)] = dblk[pl.ds(k * KS + i * 16, 16)]
                pltpu.sync_copy(g0, acc.at[d0], add=True)
                return cc

            lax.fori_loop(0, IB, chunk, 0)
            return carry

        lax.fori_loop(0, NCHUNK1 // IB, block, 0)

    plsc.subcore_barrier()

    r0 = s * ZR

    @pl.when(c == 0)
    def _():
        pltpu.sync_copy(acc.at[pl.ds(r0, ZR)], agg_out.at[pl.ds(r0, ZR)])

    @pl.when(c == 1)
    def _():
        pltpu.sync_copy(acc.at[pl.ds(r0, ZR)], deg_out.at[pl.ds(r0, ZR)])

    @pl.when((s == 0) & (c == 0))
    def _():
        pltpu.sync_copy(acc.at[pl.ds(NS * ZR, ZREM)],
                        agg_out.at[pl.ds(NS * ZR, ZREM)])

    @pl.when((s == 0) & (c == 1))
    def _():
        pltpu.sync_copy(acc.at[pl.ds(NS * ZR, ZREM)],
                        deg_out.at[pl.ds(NS * ZR, ZREM)])


_SEG_SCRATCH = (
    pltpu.VMEM((IB * KS,), jnp.int32),
    pltpu.VMEM((IB * KS,), jnp.int32),
    pltpu.VMEM((KS,), jnp.int32),
    pltpu.VMEM((KS,), jnp.int32),
    pltpu.VMEM((KS, D), jnp.float32),
    pltpu.SemaphoreType.DMA,
    pltpu.VMEM((KS, D), jnp.float32),
    pltpu.SemaphoreType.DMA,
    pltpu.VMEM_SHARED((N, D), jnp.float32),
)

_agg_deg = pl.kernel(
    _agg_deg_body,
    out_type=(jax.ShapeDtypeStruct((N, D), jnp.float32),
              jax.ShapeDtypeStruct((N, D), jnp.float32)),
    mesh=_mesh,
    scratch_types=_SEG_SCRATCH,
)


def _segsum_body(table, src1d, dst1d, zkd, agg_out,
                 sblk, dblk, d0, d1, g0, sem0, g1, sem1, acc):
    c = lax.axis_index("c")
    s = lax.axis_index("s")

    _zero_acc(zkd, g0, acc, s)
    plsc.subcore_barrier()

    base_e = (c * NS + s) * EPT
    _seg_loop(table, src1d, dst1d, acc, base_e, NCHUNKS // IB,
              sblk, dblk, d0, d1, g0, sem0, g1, sem1)

    plsc.subcore_barrier()

    out_base = c * N
    r0 = s * ZR
    pltpu.sync_copy(acc.at[pl.ds(r0, ZR)], agg_out.at[pl.ds(out_base + r0, ZR)])

    @pl.when(s == 0)
    def _():
        pltpu.sync_copy(acc.at[pl.ds(NS * ZR, ZREM)],
                        agg_out.at[pl.ds(out_base + NS * ZR, ZREM)])


_segsum = pl.kernel(
    _segsum_body,
    out_type=(jax.ShapeDtypeStruct((NC * N, D), jnp.float32),),
    mesh=_mesh,
    scratch_types=_SEG_SCRATCH,
)


def _gpair_body(A, B, u1d, v1d, g_out,
                ublk, vblk, r0, r1, r2, sa0, sa1, sa2, sb0, sb1, sb2):
    c = lax.axis_index("c")
    s = lax.axis_index("s")
    base_e = (c * NS + s) * EPT
    pltpu.sync_copy(u1d.at[pl.ds(base_e, EPT)], ublk)
    pltpu.sync_copy(v1d.at[pl.ds(base_e, EPT)], vblk)

    bufs = (r0, r1, r2)
    sas = (sa0, sa1, sa2)
    sbs = (sb0, sb1, sb2)

    def start_a(j, t):
        pltpu.async_copy(A.at[ublk.at[pl.ds(j * KG, KG)]], bufs[t], sas[t])

    def add_b(j, t):
        pltpu.make_async_copy(A.at[pl.ds(0, KG)], bufs[t], sas[t]).wait()
        pltpu.async_copy(B.at[vblk.at[pl.ds(j * KG, KG)]], bufs[t],
                         sbs[t], add=True)

    def finish(j, t):
        off = pl.multiple_of(base_e + j * KG, 8)
        pltpu.make_async_copy(B.at[pl.ds(0, KG)], bufs[t], sbs[t]).wait()
        pltpu.sync_copy(bufs[t], g_out.at[pl.ds(off, KG)])

    start_a(0, 0)
    start_a(1, 1)
    add_b(0, 0)

    def trio(m, carry):
        k = 3 * m
        for t in range(3):
            start_a(k + t + 2, (t + 2) % 3)
            add_b(k + t + 1, (t + 1) % 3)
            finish(k + t, t)
        return carry

    lax.fori_loop(0, (NCHUNKG - 2) // 3, trio, 0)
    add_b(NCHUNKG - 1, (NCHUNKG - 1) % 3)
    finish(NCHUNKG - 2, (NCHUNKG - 2) % 3)
    finish(NCHUNKG - 1, (NCHUNKG - 1) % 3)


_gpair = pl.kernel(
    _gpair_body,
    out_type=(jax.ShapeDtypeStruct((E, D), jnp.float32),),
    mesh=_mesh,
    scratch_types=(
        pltpu.VMEM((EPT,), jnp.int32),
        pltpu.VMEM((EPT,), jnp.int32),
        pltpu.VMEM((KG, D), jnp.float32),
        pltpu.VMEM((KG, D), jnp.float32),
        pltpu.VMEM((KG, D), jnp.float32),
        pltpu.SemaphoreType.DMA,
        pltpu.SemaphoreType.DMA,
        pltpu.SemaphoreType.DMA,
        pltpu.SemaphoreType.DMA,
        pltpu.SemaphoreType.DMA,
        pltpu.SemaphoreType.DMA,
    ),
)


def _dt(p, w):
    return lax.dot_general(p, w, (((1,), (1,)), ((), ())),
                           preferred_element_type=jnp.float32)


BR = 1000


def _tc1_body(a, dg, xr, wl, bl, wr, out):
    mean = a[...] / jnp.maximum(dg[:, 0:1], 1.0)
    out[...] = jnp.maximum(_dt(mean, wl[...]) + _dt(xr[...], wr[...])
                           + bl[...], 0.0)


def _tc2_body(a0, a1, dg, hr, wl, bl, wr, wu, wv, aout, bout):
    mean = (a0[...] + a1[...]) / jnp.maximum(dg[:, 0:1], 1.0)
    h = jnp.maximum(_dt(mean, wl[...]) + _dt(hr[...], wr[...]) + bl[...], 0.0)
    aout[...] = _dt(h, wu[...])
    bout[...] = _dt(h, wv[...])


def _tc_layer1(agg, deg, x, wl, bl, wr):
    return pl.pallas_call(
        _tc1_body,
        grid=(N // BR,),
        in_specs=[
            pl.BlockSpec((BR, D), lambda i: (i, 0)),
            pl.BlockSpec((BR, D), lambda i: (i, 0)),
            pl.BlockSpec((BR, D), lambda i: (i, 0)),
            pl.BlockSpec((D, D), lambda i: (0, 0)),
            pl.BlockSpec((1, D), lambda i: (0, 0)),
            pl.BlockSpec((D, D), lambda i: (0, 0)),
        ],
        out_specs=pl.BlockSpec((BR, D), lambda i: (i, 0)),
        out_shape=jax.ShapeDtypeStruct((N, D), jnp.float32),
    )(agg, deg, x, wl, bl, wr)


def _tc_layer2(aggp, deg, h1, wl, bl, wr, wu, wv):
    return pl.pallas_call(
        _tc2_body,
        grid=(N // BR,),
        in_specs=[
            pl.BlockSpec((BR, D), lambda i: (i, 0)),
            pl.BlockSpec((BR, D), lambda i: (i + N // BR, 0)),
            pl.BlockSpec((BR, D), lambda i: (i, 0)),
            pl.BlockSpec((BR, D), lambda i: (i, 0)),
            pl.BlockSpec((D, D), lambda i: (0, 0)),
            pl.BlockSpec((1, D), lambda i: (0, 0)),
            pl.BlockSpec((D, D), lambda i: (0, 0)),
            pl.BlockSpec((D, D), lambda i: (0, 0)),
            pl.BlockSpec((D, D), lambda i: (0, 1)),
        ],
        out_specs=[pl.BlockSpec((BR, D), lambda i: (i, 0)),
                   pl.BlockSpec((BR, D), lambda i: (i, 0))],
        out_shape=[jax.ShapeDtypeStruct((N, D), jnp.float32),
                   jax.ShapeDtypeStruct((N, D), jnp.float32)],
    )(aggp, aggp, deg, h1, wl, bl, wr, wu, wv)


BE = 16000


RB = BE // 128


def _tc3_body(g, at, le, w1e, b1, w2, b2, out):
    hmid = jnp.maximum(g[...] + _dt(at[...], w1e[...]) + b1[...], 0.0)
    h3 = hmid.reshape(RB, 128, D)
    s = lax.dot_general(h3, w2[...], (((2,), (0,)), ((), ())),
                        preferred_element_type=jnp.float32)
    out[...] = (s + le[0] + b2[0]).reshape(1, RB, 128)


def _tc_edge(g, attr, le, w1e, b1, w2, b2):
    return pl.pallas_call(
        _tc3_body,
        grid=(E // BE,),
        in_specs=[
            pl.BlockSpec((BE, D), lambda i: (i, 0)),
            pl.BlockSpec((BE, DE), lambda i: (i, 0)),
            pl.BlockSpec((1, RB, 128), lambda i: (i, 0, 0)),
            pl.BlockSpec((D, DE), lambda i: (0, 0)),
            pl.BlockSpec((1, D), lambda i: (0, 0)),
            pl.BlockSpec((D,), lambda i: (0,)),
            pl.BlockSpec(memory_space=pltpu.SMEM),
        ],
        out_specs=pl.BlockSpec((1, RB, 128), lambda i: (i, 0, 0)),
        out_shape=jax.ShapeDtypeStruct((E // BE, RB, 128), jnp.float32),
    )(g, attr, le, w1e, b1, w2, b2)


def kernel(x, edge_index, edge_u, edge_v, edge_attr, log_exposure,
           W1_l, b1_l, W1_r, W2_l, b2_l, W2_r, mW1, mb1, mW2, mb2):
    src = edge_index[0].astype(jnp.int32)
    dst = edge_index[1].astype(jnp.int32)
    u = edge_u.astype(jnp.int32)
    v = edge_v.astype(jnp.int32)
    zkd = jnp.zeros((KS, D), jnp.float32)
    onesd = jnp.ones((KS, D), jnp.float32)

    agg, deg = _agg_deg(x, src, dst, zkd, onesd)
    h1 = _tc_layer1(agg, deg, x, W1_l, b1_l.reshape(1, D), W1_r)
    (aggp2,) = _segsum(h1, src, dst, zkd)
    A, B = _tc_layer2(aggp2, deg, h1, W2_l, b2_l.reshape(1, D), W2_r,
                      mW1, mW1)
    (g,) = _gpair(A, B, u, v)
    out = _tc_edge(g, edge_attr, log_exposure.reshape(E // BE, RB, 128),
                   mW1[:, 2 * D:], mb1.reshape(1, D), mW2.reshape(D),
                   mb2)
    return out.reshape(E)
